# baseline (speedup 1.0000x reference)
import functools

import jax
import jax.numpy as jnp
from jax import lax
from jax.experimental import pallas as pl
from jax.experimental.pallas import tpu as pltpu
from jax.experimental.pallas import tpu_sc as plsc

N_R = 10000
N_M = 50000
D = 128
EMB = 1024
L = 4
E = 200000

NR_PAD = 10240
NM_PAD = 50176
E_PAD = 200704
EBLK = E_PAD // 1024

NC = 2
NS = 16
ET = E_PAD // NS
G = 128
REG = 204800
PK = 131072

C_R = NR_PAD // 2
C_M = NM_PAD // 8

_f32 = jnp.float32
_i32 = jnp.int32


def _prefix_8x128(m):
  li = lax.broadcasted_iota(_i32, (8, 128), 1)
  x = m
  for k in (1, 2, 4, 8, 16, 32, 64):
    x = x + jnp.where(li >= k, pltpu.roll(x, k, 1), 0)
  rt = x[:, 127:128]
  si = lax.broadcasted_iota(_i32, (8, 1), 0)
  r = rt
  for k in (1, 2, 4):
    r = r + jnp.where(si >= k, pltpu.roll(r, k, 0), 0)
  return x + (r - rt)


def _make_rank(nchunk, C):
  nreg = nchunk + 1

  def body(d_ref, rank_ref, cnt_ref, carry):
    pid = pl.program_id(0)

    @pl.when(pid == 0)
    def _():
      for k in range(nreg):
        carry[k] = 0

    d = d_ref[0]
    rank = jnp.zeros((8, 128), _i32)
    for k in range(nreg):
      if k < nchunk:
        mk = (d >= k * C) & (d < (k + 1) * C)
      else:
        mk = d < 0
      mi = jnp.where(mk, 1, 0).astype(_i32)
      incl = _prefix_8x128(mi)
      ck = carry[k]
      rank = rank + jnp.where(mk, k * REG + ck + (incl - mi), 0)
      carry[k] = ck + jnp.sum(mi)
    rank_ref[0] = rank

    si = lax.broadcasted_iota(_i32, (8, 128), 0)
    li = lax.broadcasted_iota(_i32, (8, 128), 1)
    cvals = jnp.zeros((8, 128), _i32)
    for k in range(nreg):
      cvals = cvals + jnp.where((si == k) & (li == 0), carry[k], 0)
    cnt_ref[...] = cvals

  def call(d3):
    return pl.pallas_call(
        body,
        grid=(EBLK,),
        in_specs=[pl.BlockSpec((1, 8, 128), lambda i: (i, 0, 0))],
        out_specs=[pl.BlockSpec((1, 8, 128), lambda i: (i, 0, 0)),
                   pl.BlockSpec((8, 128), lambda i: (0, 0))],
        out_shape=[jax.ShapeDtypeStruct((EBLK, 8, 128), _i32),
                   jax.ShapeDtypeStruct((8, 128), _i32)],
        scratch_shapes=[pltpu.SMEM((8,), _i32)],
    )(d3)

  return call


def _make_permute():
  mesh = plsc.VectorSubcoreMesh(core_axis_name="c", subcore_axis_name="s",
                                num_cores=NC, num_subcores=NS)
  nrow = ET // G

  PB = 2048

  @functools.partial(
      pl.kernel,
      out_type=(jax.ShapeDtypeStruct((2 * REG,), _i32),
                jax.ShapeDtypeStruct((8 * REG,), _i32),
                jax.ShapeDtypeStruct((8 * REG,), _i32)),
      mesh=mesh,
      scratch_types=[
          pltpu.VMEM((ET,), _i32),
          pltpu.VMEM((ET,), _i32),
          pltpu.VMEM((nrow, G), _i32),
          pltpu.VMEM((nrow, G), _i32),
          pltpu.VMEM((nrow, G), _i32),
          pltpu.VMEM((PB,), _i32),
          pltpu.VMEM_SHARED((4 * REG + PB,), _i32),
          pltpu.SemaphoreType.DMA,
      ],
  )
  def k(s1, d1, r1, s2, d2, r2, s3, d3, r3, o1, o2, o3,
        src_v, dst_v, rank2, val2, sidx, padb, spbuf, sem):
    c = lax.axis_index("c")
    s = lax.axis_index("s")
    iota16 = lax.iota(_i32, 16)

    def phase(s_hbm, d_hbm, r3_hbm, out_hbm, nchunk, C):
      npc = nchunk // NC
      own = npc * REG
      base_c = c * own
      padval = jnp.full((16,), C * PK, _i32)

      def pfill(i, _):
        padb[pl.ds(i * 16, 16)] = padval
        return 0
      lax.fori_loop(0, PB // 16, pfill, 0)
      shf = own // NS
      def sfill(f, _):
        pltpu.async_copy(padb, spbuf.at[pl.ds(s * shf + f * PB, PB)], sem)
        return 0
      lax.fori_loop(0, shf // PB, sfill, 0)
      def sfill_d(f, _):
        pltpu.make_async_copy(padb, spbuf.at[pl.ds(s * shf + f * PB, PB)],
                              sem).wait()
        return 0
      lax.fori_loop(0, shf // PB, sfill_d, 0)
      if shf % PB:
        pltpu.sync_copy(padb.at[pl.ds(0, shf % PB)],
                        spbuf.at[pl.ds(s * shf + (shf // PB) * PB,
                                       shf % PB)])

      pltpu.sync_copy(s_hbm.at[pl.ds(s * ET, ET)], src_v)
      pltpu.sync_copy(d_hbm.at[pl.ds(s * ET, ET)], dst_v)
      pltpu.sync_copy(r3_hbm.at[s], rank2)

      def pack(i, _):
        r = i // 8
        cc = (i % 8) * 16
        sv = src_v[pl.ds(i * 16, 16)]
        dv = dst_v[pl.ds(i * 16, 16)]
        rk = rank2[r, pl.ds(cc, 16)]
        kk = jnp.zeros((16,), _i32)
        for q in range(1, nchunk):
          kk = kk + jnp.where(dv >= q * C, 1, 0)
        doff = jnp.where(dv >= 0, dv - kk * C, C)
        val2[r, pl.ds(cc, 16)] = sv + doff * PK
        mine = (rk >= base_c) & (rk < base_c + own)
        tr = 4 * REG + lax.bitwise_and(i * 16 + iota16, PB - 1)
        sidx[r, pl.ds(cc, 16)] = jnp.where(mine, rk - base_c, tr)
        return 0
      lax.fori_loop(0, ET // 16, pack, 0)
      plsc.subcore_barrier()

      def scat(j, _):
        pltpu.async_copy(val2.at[j], spbuf.at[sidx.at[j]], sem)
        return 0
      lax.fori_loop(0, nrow, scat, 0)
      def drain(j, _):
        pltpu.make_async_copy(val2.at[j], spbuf.at[sidx.at[j]], sem).wait()
        return 0
      lax.fori_loop(0, nrow, drain, 0)
      plsc.subcore_barrier()

      pltpu.sync_copy(spbuf.at[pl.ds(s * shf, shf)],
                      out_hbm.at[pl.ds(base_c + s * shf, shf)])
      plsc.subcore_barrier()

    phase(s1, d1, r1, o1, 2, C_R)
    phase(s2, d2, r2, o2, 8, C_M)
    phase(s3, d3, r3, o3, 8, C_M)

  return k


def _make_seg3():
  mesh = plsc.VectorSubcoreMesh(core_axis_name="c", subcore_axis_name="s",
                                num_cores=NC, num_subcores=NS)

  SH = 98

  @functools.partial(
      pl.kernel,
      out_type=(jax.ShapeDtypeStruct((NR_PAD, D), _f32),
                jax.ShapeDtypeStruct((NM_PAD, D), _f32),
                jax.ShapeDtypeStruct((NM_PAD, D), _f32)),
      mesh=mesh,
      scratch_types=[
          pltpu.VMEM((7 * 2048,), _i32),
          pltpu.VMEM((SH, G), _i32),
          pltpu.VMEM((3, G, D), _f32),
          pltpu.VMEM((16, D), _f32),
          pltpu.VMEM((8, 128), _i32),
          pltpu.VMEM_SHARED((C_M + 8, D), _f32),
          pltpu.SemaphoreType.DMA,
          pltpu.SemaphoreType.DMA,
          pltpu.SemaphoreType.DMA,
          pltpu.SemaphoreType.DMA,
          pltpu.SemaphoreType.DMA,
          pltpu.SemaphoreType.DMA,
          pltpu.SemaphoreType.DMA,
          pltpu.SemaphoreType.DMA,
      ],
  )
  def k(xm_hbm, xr_hbm, p1_hbm, c1_hbm, p2_hbm, c2_hbm, p3_hbm, c3_hbm,
        o1_hbm, o2_hbm, o3_hbm,
        csrc, cdst, rows_v, zb_v, cnt2, acc_sp, sem0, sem1,
        g0, g1, g2, t0, t1, t2):
    c = lax.axis_index("c")
    s = lax.axis_index("s")

    def zb_body(i, _):
      r = i // 8
      col = (i % 8) * 16
      zb_v[r, pl.ds(col, 16)] = jnp.zeros((16,), _f32)
      return 0
    lax.fori_loop(0, 16 * 8, zb_body, 0)

    def one_pass(table_hbm, packed_hbm, out_hbm, npass, C, rpt, n64, rem, p):
      kk = c * npass + p
      lo = kk * C

      nz = rpt // 16
      rz = rpt - nz * 16
      def z_body(i, _):
        off = pl.multiple_of(s * rpt + i * 16, 8)
        pltpu.async_copy(zb_v, acc_sp.at[pl.ds(off, 16)], sem0)
        return 0
      lax.fori_loop(0, nz, z_body, 0)
      if rz:
        off = pl.multiple_of(s * rpt + nz * 16, 8)
        pltpu.async_copy(zb_v.at[pl.ds(0, rz)], acc_sp.at[pl.ds(off, rz)],
                         sem0)
      def z_drain(i, _):
        off = pl.multiple_of(s * rpt + i * 16, 8)
        pltpu.make_async_copy(zb_v, acc_sp.at[pl.ds(off, 16)], sem0).wait()
        return 0
      lax.fori_loop(0, nz, z_drain, 0)
      if rz:
        off = pl.multiple_of(s * rpt + nz * 16, 8)
        pltpu.make_async_copy(zb_v.at[pl.ds(0, rz)],
                              acc_sp.at[pl.ds(off, rz)], sem0).wait()
      plsc.subcore_barrier()

      ck = cnt2[kk, pl.ds(0, 16)][0]
      nblk = lax.shift_right_logical(ck + (G - 1), 7)
      share = lax.shift_right_logical(nblk + (NS - 1), 4)

      wbase = kk * REG + s * share * G
      nst = lax.shift_right_logical(share + 15, 4)
      def st_body(i, _):
        off = pl.multiple_of(wbase + i * 2048, 8)
        pltpu.async_copy(packed_hbm.at[pl.ds(off, 2048)],
                         csrc.at[pl.ds(i * 2048, 2048)], sem1)
        return 0
      lax.fori_loop(0, nst, st_body, 0)
      def st_drain(i, _):
        off = pl.multiple_of(wbase + i * 2048, 8)
        pltpu.make_async_copy(packed_hbm.at[pl.ds(off, 2048)],
                              csrc.at[pl.ds(i * 2048, 2048)], sem1).wait()
        return 0
      lax.fori_loop(0, nst, st_drain, 0)

      def up_body(i, _):
        r = i // 8
        cu = (i % 8) * 16
        v = csrc[pl.ds(i * 16, 16)]
        csrc[pl.ds(i * 16, 16)] = lax.bitwise_and(v, PK - 1)
        cdst[r, pl.ds(cu, 16)] = lax.shift_right_logical(v, 17)
        return 0
      lax.fori_loop(0, share * 8, up_body, 0)

      gsem = (g0, g1, g2)
      tsem = (t0, t1, t2)
      def fireg(j, u):
        pltpu.async_copy(table_hbm.at[csrc.at[pl.ds(j * G, G)]],
                         rows_v.at[u], gsem[u])
      def waitg(j, u):
        pltpu.make_async_copy(table_hbm.at[csrc.at[pl.ds(j * G, G)]],
                              rows_v.at[u], gsem[u]).wait()
      def fires(j, u):
        pltpu.async_copy(rows_v.at[u], acc_sp.at[cdst.at[j]], tsem[u],
                         add=True)
      def waits(j, u):
        pltpu.make_async_copy(rows_v.at[u], acc_sp.at[cdst.at[j]],
                              tsem[u]).wait()

      def pip(tt, _):
        for u in range(3):
          j = tt * 3 + u
          um = (u + 2) % 3
          @pl.when((j >= 3) & (j - 3 < share))
          def _():
            waits(j - 3, u)
          @pl.when(j < share)
          def _():
            fireg(j, u)
          @pl.when((j >= 1) & (j - 1 < share))
          def _():
            waitg(j - 1, um)
            fires(j - 1, um)
        return 0
      lax.fori_loop(0, lax.div(share + 5, jnp.int32(3)), pip, 0)
      plsc.subcore_barrier()

      def w_body(i, _):
        aoff = pl.multiple_of(s * rpt + i * 64, 8)
        ooff = pl.multiple_of(lo + s * rpt + i * 64, 8)
        pltpu.async_copy(acc_sp.at[pl.ds(aoff, 64)],
                         out_hbm.at[pl.ds(ooff, 64)], sem0)
        return 0
      lax.fori_loop(0, n64, w_body, 0)
      if rem:
        aoff = pl.multiple_of(s * rpt + n64 * 64, 8)
        ooff = pl.multiple_of(lo + s * rpt + n64 * 64, 8)
        pltpu.async_copy(acc_sp.at[pl.ds(aoff, rem)],
                         out_hbm.at[pl.ds(ooff, rem)], sem0)
      def w_drain(i, _):
        aoff = pl.multiple_of(s * rpt + i * 64, 8)
        ooff = pl.multiple_of(lo + s * rpt + i * 64, 8)
        pltpu.make_async_copy(acc_sp.at[pl.ds(aoff, 64)],
                              out_hbm.at[pl.ds(ooff, 64)], sem0).wait()
        return 0
      lax.fori_loop(0, n64, w_drain, 0)
      if rem:
        aoff = pl.multiple_of(s * rpt + n64 * 64, 8)
        ooff = pl.multiple_of(lo + s * rpt + n64 * 64, 8)
        pltpu.make_async_copy(acc_sp.at[pl.ds(aoff, rem)],
                              out_hbm.at[pl.ds(ooff, rem)], sem0).wait()
      plsc.subcore_barrier()

    def agg(table_hbm, packed_hbm, counts_hbm, out_hbm, nchunk, C):
      pltpu.sync_copy(counts_hbm, cnt2)
      npass = nchunk // 2
      rpt = C // NS
      n64 = rpt // 64
      rem = rpt - n64 * 64
      for p in range(npass):
        one_pass(table_hbm, packed_hbm, out_hbm, npass, C, rpt, n64, rem, p)

    agg(xm_hbm, p1_hbm, c1_hbm, o1_hbm, 2, C_R)
    agg(xr_hbm, p2_hbm, c2_hbm, o2_hbm, 8, C_M)
    agg(xm_hbm, p3_hbm, c3_hbm, o3_hbm, 8, C_M)

  return k


def _make_cnt(n_dst_pad, nchunk, C):
  npass = nchunk // 2
  rpt = C // NS
  SH = 98
  mesh = plsc.VectorSubcoreMesh(core_axis_name="c", subcore_axis_name="s",
                                num_cores=NC, num_subcores=NS)

  @functools.partial(
      pl.kernel,
      out_type=jax.ShapeDtypeStruct((n_dst_pad,), _f32),
      mesh=mesh,
      scratch_types=[
          pltpu.VMEM((7 * 2048,), _i32),
          pltpu.VMEM((SH, G), _i32),
          pltpu.VMEM((G,), _f32),
          pltpu.VMEM((2048,), _f32),
          pltpu.VMEM((2048,), _f32),
          pltpu.VMEM((8, 128), _i32),
          pltpu.VMEM_SHARED((C + 8,), _f32),
          pltpu.SemaphoreType.DMA,
          pltpu.SemaphoreType.DMA,
      ],
  )
  def k(packed_hbm, counts_hbm, out_hbm,
        pk_v, cdst, ones_v, zb_v, wb_v, cnt2, acc_sp, sem0, sem1):
    c = lax.axis_index("c")
    s = lax.axis_index("s")
    pltpu.sync_copy(counts_hbm, cnt2)
    def zi(i, _):
      zb_v[pl.ds(i * 16, 16)] = jnp.zeros((16,), _f32)
      return 0
    lax.fori_loop(0, 2048 // 16, zi, 0)
    def oi(i, _):
      ones_v[pl.ds(i * 16, 16)] = jnp.ones((16,), _f32)
      return 0
    lax.fori_loop(0, G // 16, oi, 0)

    nw = NS
    wrt = C // nw
    while wrt % 16:
      nw //= 2
      wrt = C // nw

    for p in range(npass):
      kk = c * npass + p
      lo = kk * C
      off0 = pl.multiple_of(s * wrt, 16)
      @pl.when(s < nw)
      def _():
        pltpu.sync_copy(zb_v.at[pl.ds(0, wrt)], acc_sp.at[pl.ds(off0, wrt)])
      plsc.subcore_barrier()

      ck = cnt2[kk, pl.ds(0, 16)][0]
      nblk = lax.shift_right_logical(ck + (G - 1), 7)
      share = lax.shift_right_logical(nblk + (NS - 1), 4)
      wbase = kk * REG + s * share * G
      nst = lax.shift_right_logical(share + 15, 4)
      def st_body(i, _):
        off = pl.multiple_of(wbase + i * 2048, 8)
        pltpu.async_copy(packed_hbm.at[pl.ds(off, 2048)],
                         pk_v.at[pl.ds(i * 2048, 2048)], sem1)
        return 0
      lax.fori_loop(0, nst, st_body, 0)
      def st_drain(i, _):
        off = pl.multiple_of(wbase + i * 2048, 8)
        pltpu.make_async_copy(packed_hbm.at[pl.ds(off, 2048)],
                              pk_v.at[pl.ds(i * 2048, 2048)], sem1).wait()
        return 0
      lax.fori_loop(0, nst, st_drain, 0)
      def up_body(i, _):
        r = i // 8
        cu = (i % 8) * 16
        v = pk_v[pl.ds(i * 16, 16)]
        cdst[r, pl.ds(cu, 16)] = lax.shift_right_logical(v, 17)
        return 0
      lax.fori_loop(0, share * 8, up_body, 0)
      def cb(b, _):
        pltpu.sync_copy(ones_v, acc_sp.at[cdst.at[b]], add=True)
        return 0
      lax.fori_loop(0, share, cb, 0)
      plsc.subcore_barrier()
      ooff = pl.multiple_of(lo + s * wrt, 16)
      @pl.when(s < nw)
      def _():
        pltpu.sync_copy(acc_sp.at[pl.ds(off0, wrt)], wb_v.at[pl.ds(0, wrt)])
        pltpu.sync_copy(wb_v.at[pl.ds(0, wrt)], out_hbm.at[pl.ds(ooff, wrt)])
      plsc.subcore_barrier()

  return k


_sc_cache = {}


def _sc_kernels():
  if "k" not in _sc_cache:
    _sc_cache["k"] = (
        _make_rank(2, C_R),
        _make_rank(8, C_M),
        _make_permute(),
        _make_seg3(),
        _make_cnt(NR_PAD, 2, C_R),
        _make_cnt(NM_PAD, 8, C_M),
    )
  return _sc_cache["k"]


_BLK = 1024


def _dense_r_body(s_ref, cnt_ref, x_ref, wn_ref, wr_ref, b_ref, o_ref):
  agg = s_ref[...] / jnp.maximum(cnt_ref[...], 1.0)
  o_ref[...] = (jnp.dot(agg, wn_ref[...], preferred_element_type=_f32)
                + jnp.dot(x_ref[...], wr_ref[...], preferred_element_type=_f32)
                + b_ref[0:1, :])


def _dense_r(n_pad, s, cnt, x, wn, wr, b):
  grid = (n_pad // _BLK,)
  return pl.pallas_call(
      _dense_r_body,
      grid=grid,
      in_specs=[
          pl.BlockSpec((_BLK, D), lambda i: (i, 0)),
          pl.BlockSpec((_BLK, 1), lambda i: (i, 0)),
          pl.BlockSpec((_BLK, D), lambda i: (i, 0)),
          pl.BlockSpec((D, D), lambda i: (0, 0)),
          pl.BlockSpec((D, D), lambda i: (0, 0)),
          pl.BlockSpec((8, D), lambda i: (0, 0)),
      ],
      out_specs=pl.BlockSpec((_BLK, D), lambda i: (i, 0)),
      out_shape=jax.ShapeDtypeStruct((n_pad, D), _f32),
  )(s, cnt, x, wn, wr, b)


def _dense_m_body(s1_ref, c1_ref, s2_ref, c2_ref, x_ref,
                  wn1_ref, wr1_ref, b1_ref, wn2_ref, wr2_ref, b2_ref, o_ref):
  a1 = s1_ref[...] / jnp.maximum(c1_ref[...], 1.0)
  a2 = s2_ref[...] / jnp.maximum(c2_ref[...], 1.0)
  x = x_ref[...]
  o_ref[...] = (jnp.dot(a1, wn1_ref[...], preferred_element_type=_f32)
                + jnp.dot(x, wr1_ref[...] + wr2_ref[...],
                          preferred_element_type=_f32)
                + jnp.dot(a2, wn2_ref[...], preferred_element_type=_f32)
                + (b1_ref[0:1, :] + b2_ref[0:1, :]))


def _dense_m(n_pad, s1, c1, s2, c2, x, wn1, wr1, b1, wn2, wr2, b2):
  grid = (n_pad // _BLK,)
  row = lambda i: (i, 0)
  full = lambda i: (0, 0)
  return pl.pallas_call(
      _dense_m_body,
      grid=grid,
      in_specs=[
          pl.BlockSpec((_BLK, D), row),
          pl.BlockSpec((_BLK, 1), row),
          pl.BlockSpec((_BLK, D), row),
          pl.BlockSpec((_BLK, 1), row),
          pl.BlockSpec((_BLK, D), row),
          pl.BlockSpec((D, D), full),
          pl.BlockSpec((D, D), full),
          pl.BlockSpec((8, D), full),
          pl.BlockSpec((D, D), full),
          pl.BlockSpec((D, D), full),
          pl.BlockSpec((8, D), full),
      ],
      out_specs=pl.BlockSpec((_BLK, D), row),
      out_shape=jax.ShapeDtypeStruct((n_pad, D), _f32),
  )(s1, c1, s2, c2, x, wn1, wr1, b1, wn2, wr2, b2)


def _head_body(x_ref, emb_ref, ty_ref, wlr_ref, blr_ref,
               w0_ref, w1_ref, bt_ref, o_ref):
  ro = jnp.dot(x_ref[...], wlr_ref[...], preferred_element_type=_f32) \
      + blr_ref[0:1, :]
  emb = emb_ref[...]
  o0 = jnp.dot(emb, w0_ref[...], preferred_element_type=_f32) + bt_ref[0:1, :]
  o1 = jnp.dot(emb, w1_ref[...], preferred_element_type=_f32) + bt_ref[1:2, :]
  sel = jnp.where(ty_ref[...] == 0, o0, o1)
  dot = jnp.sum(ro * sel, axis=1)
  na = jnp.sqrt(jnp.sum(ro * ro, axis=1))
  nb = jnp.sqrt(jnp.sum(sel * sel, axis=1))
  cos = dot / (jnp.maximum(na, 1e-8) * jnp.maximum(nb, 1e-8))
  o_ref[...] = (cos + 1.0) * 0.5


def _head(x, emb, ty, wlr, blr, w0, w1, bt):
  grid = (NR_PAD // _BLK,)
  row = lambda i: (i, 0)
  full = lambda i: (0, 0)
  return pl.pallas_call(
      _head_body,
      grid=grid,
      in_specs=[
          pl.BlockSpec((_BLK, D), row),
          pl.BlockSpec((_BLK, EMB), row),
          pl.BlockSpec((_BLK, 1), row),
          pl.BlockSpec((D, D), full),
          pl.BlockSpec((8, D), full),
          pl.BlockSpec((EMB, D), full),
          pl.BlockSpec((EMB, D), full),
          pl.BlockSpec((8, D), full),
      ],
      out_specs=pl.BlockSpec((_BLK,), lambda i: (i,)),
      out_shape=jax.ShapeDtypeStruct((NR_PAD,), _f32),
  )(x, emb, ty, wlr, blr, w0, w1, bt)


def _pad_bias(b):
  return jnp.pad(b.reshape(1, D), ((0, 7), (0, 0)))


def kernel(x_reaction, x_molecule, ei_m2r, ei_r2m, ei_m2m, output_notes_opt,
           output_nodes_types, Wn, Wr, bconv, W_lr, b_lr, W_types, b_types):
  x_r = jnp.pad(x_reaction, ((0, NR_PAD - N_R), (0, 0)))
  x_m = jnp.pad(x_molecule, ((0, NM_PAD - N_M), (0, 0)))

  def split(ei):
    src = jnp.pad(ei[0].astype(_i32), (0, E_PAD - E))
    dst = jnp.pad(ei[1].astype(_i32), (0, E_PAD - E), constant_values=-1)
    return src, dst

  (_rank_r, _rank_m, _permute, _seg3, _cntk_r, _cntk_m) = _sc_kernels()

  s_m2r, d_m2r = split(ei_m2r)
  s_r2m, d_r2m = split(ei_r2m)
  s_m2m, d_m2m = split(ei_m2m)

  rk1, ct1 = _rank_r(d_m2r.reshape(EBLK, 8, 128))
  rk2, ct2 = _rank_m(d_r2m.reshape(EBLK, 8, 128))
  rk3, ct3 = _rank_m(d_m2m.reshape(EBLK, 8, 128))

  p1, p2, p3 = _permute(
      s_m2r, d_m2r, rk1.reshape(NS, ET // G, G),
      s_r2m, d_r2m, rk2.reshape(NS, ET // G, G),
      s_m2m, d_m2m, rk3.reshape(NS, ET // G, G))

  cnt_r = _cntk_r(p1, ct1).reshape(NR_PAD, 1)
  cnt_m1 = _cntk_m(p2, ct2).reshape(NM_PAD, 1)
  cnt_m2 = _cntk_m(p3, ct3).reshape(NM_PAD, 1)

  for l in range(L):
    agg_r, agg_m1, agg_m2 = _seg3(x_m, x_r, p1, ct1, p2, ct2, p3, ct3)
    x_r_new = _dense_r(NR_PAD, agg_r, cnt_r, x_r,
                       Wn[l, 0], Wr[l, 0], _pad_bias(bconv[l, 0]))
    x_m_new = _dense_m(NM_PAD, agg_m1, cnt_m1, agg_m2, cnt_m2, x_m,
                       Wn[l, 1], Wr[l, 1], _pad_bias(bconv[l, 1]),
                       Wn[l, 2], Wr[l, 2], _pad_bias(bconv[l, 2]))
    x_r, x_m = x_r_new, x_m_new

  emb = jnp.pad(output_notes_opt, ((0, NR_PAD - N_R), (0, 0)))
  ty = jnp.pad(output_nodes_types.astype(_i32),
               (0, NR_PAD - N_R)).reshape(NR_PAD, 1)
  bt = jnp.pad(b_types, ((0, 6), (0, 0)))
  out = _head(x_r, emb, ty, W_lr, _pad_bias(b_lr),
              W_types[0], W_types[1], bt)
  return out[:N_R]


# speedup vs baseline: 1.1895x; 1.1895x over previous
import functools

import jax
import jax.numpy as jnp
from jax import lax
from jax.experimental import pallas as pl
from jax.experimental.pallas import tpu as pltpu
from jax.experimental.pallas import tpu_sc as plsc

N_R = 10000
N_M = 50000
D = 128
EMB = 1024
L = 4
E = 200000

NR_PAD = 10240
NM_PAD = 50176
E_PAD = 200704
EBLK = E_PAD // 1024

NC = 2
NS = 16
ET = E_PAD // NS
G = 128
REG = 204800
PK = 131072

C_R = NR_PAD // 2
C_M = NM_PAD // 8

_f32 = jnp.float32
_i32 = jnp.int32


def _prefix_8x128(m):
  li = lax.broadcasted_iota(_i32, (8, 128), 1)
  x = m
  for k in (1, 2, 4, 8, 16, 32, 64):
    x = x + jnp.where(li >= k, pltpu.roll(x, k, 1), 0)
  rt = x[:, 127:128]
  si = lax.broadcasted_iota(_i32, (8, 1), 0)
  r = rt
  for k in (1, 2, 4):
    r = r + jnp.where(si >= k, pltpu.roll(r, k, 0), 0)
  return x + (r - rt)


def _make_rank(nchunk, C):
  nreg = nchunk + 1

  def body(d_ref, rank_ref, cnt_ref, carry):
    pid = pl.program_id(0)

    @pl.when(pid == 0)
    def _():
      for k in range(nreg):
        carry[k] = 0

    d = d_ref[0]
    rank = jnp.zeros((8, 128), _i32)
    for k in range(nreg):
      if k < nchunk:
        mk = (d >= k * C) & (d < (k + 1) * C)
      else:
        mk = d < 0
      mi = jnp.where(mk, 1, 0).astype(_i32)
      incl = _prefix_8x128(mi)
      ck = carry[k]
      rank = rank + jnp.where(mk, k * REG + ck + (incl - mi), 0)
      carry[k] = ck + jnp.sum(mi)
    rank_ref[0] = rank

    si = lax.broadcasted_iota(_i32, (8, 128), 0)
    li = lax.broadcasted_iota(_i32, (8, 128), 1)
    cvals = jnp.zeros((8, 128), _i32)
    for k in range(nreg):
      cvals = cvals + jnp.where((si == k) & (li == 0), carry[k], 0)
    cnt_ref[...] = cvals

  def call(d3):
    return pl.pallas_call(
        body,
        grid=(EBLK,),
        in_specs=[pl.BlockSpec((1, 8, 128), lambda i: (i, 0, 0))],
        out_specs=[pl.BlockSpec((1, 8, 128), lambda i: (i, 0, 0)),
                   pl.BlockSpec((8, 128), lambda i: (0, 0))],
        out_shape=[jax.ShapeDtypeStruct((EBLK, 8, 128), _i32),
                   jax.ShapeDtypeStruct((8, 128), _i32)],
        scratch_shapes=[pltpu.SMEM((8,), _i32)],
    )(d3)

  return call


def _make_permute():
  mesh = plsc.VectorSubcoreMesh(core_axis_name="c", subcore_axis_name="s",
                                num_cores=NC, num_subcores=NS)
  nrow = ET // G

  PB = 2048

  @functools.partial(
      pl.kernel,
      out_type=(jax.ShapeDtypeStruct((2 * REG,), _i32),
                jax.ShapeDtypeStruct((8 * REG,), _i32),
                jax.ShapeDtypeStruct((8 * REG,), _i32)),
      mesh=mesh,
      scratch_types=[
          pltpu.VMEM((ET,), _i32),
          pltpu.VMEM((ET,), _i32),
          pltpu.VMEM((nrow, G), _i32),
          pltpu.VMEM((nrow, G), _i32),
          pltpu.VMEM((nrow, G), _i32),
          pltpu.VMEM((PB,), _i32),
          pltpu.VMEM_SHARED((4 * REG + PB,), _i32),
          pltpu.SemaphoreType.DMA,
      ],
  )
  def k(s1, d1, r1, s2, d2, r2, s3, d3, r3, o1, o2, o3,
        src_v, dst_v, rank2, val2, sidx, padb, spbuf, sem):
    c = lax.axis_index("c")
    s = lax.axis_index("s")
    iota16 = lax.iota(_i32, 16)

    def phase(s_hbm, d_hbm, r3_hbm, out_hbm, nchunk, C):
      npc = nchunk // NC
      own = npc * REG
      base_c = c * own
      padval = jnp.full((16,), C * PK, _i32)

      def pfill(i, _):
        padb[pl.ds(i * 16, 16)] = padval
        return 0
      lax.fori_loop(0, PB // 16, pfill, 0)
      shf = own // NS
      def sfill(f, _):
        pltpu.async_copy(padb, spbuf.at[pl.ds(s * shf + f * PB, PB)], sem)
        return 0
      lax.fori_loop(0, shf // PB, sfill, 0)
      def sfill_d(f, _):
        pltpu.make_async_copy(padb, spbuf.at[pl.ds(s * shf + f * PB, PB)],
                              sem).wait()
        return 0
      lax.fori_loop(0, shf // PB, sfill_d, 0)
      if shf % PB:
        pltpu.sync_copy(padb.at[pl.ds(0, shf % PB)],
                        spbuf.at[pl.ds(s * shf + (shf // PB) * PB,
                                       shf % PB)])

      pltpu.sync_copy(s_hbm.at[pl.ds(s * ET, ET)], src_v)
      pltpu.sync_copy(d_hbm.at[pl.ds(s * ET, ET)], dst_v)
      pltpu.sync_copy(r3_hbm.at[s], rank2)

      def pack(i, _):
        r = i // 8
        cc = (i % 8) * 16
        sv = src_v[pl.ds(i * 16, 16)]
        dv = dst_v[pl.ds(i * 16, 16)]
        rk = rank2[r, pl.ds(cc, 16)]
        kk = jnp.zeros((16,), _i32)
        for q in range(1, nchunk):
          kk = kk + jnp.where(dv >= q * C, 1, 0)
        doff = jnp.where(dv >= 0, dv - kk * C, C)
        val2[r, pl.ds(cc, 16)] = sv + doff * PK
        mine = (rk >= base_c) & (rk < base_c + own)
        tr = 4 * REG + lax.bitwise_and(i * 16 + iota16, PB - 1)
        sidx[r, pl.ds(cc, 16)] = jnp.where(mine, rk - base_c, tr)
        return 0
      lax.fori_loop(0, ET // 16, pack, 0)
      plsc.subcore_barrier()

      def scat(j, _):
        pltpu.async_copy(val2.at[j], spbuf.at[sidx.at[j]], sem)
        return 0
      lax.fori_loop(0, nrow, scat, 0)
      def drain(j, _):
        pltpu.make_async_copy(val2.at[j], spbuf.at[sidx.at[j]], sem).wait()
        return 0
      lax.fori_loop(0, nrow, drain, 0)
      plsc.subcore_barrier()

      pltpu.sync_copy(spbuf.at[pl.ds(s * shf, shf)],
                      out_hbm.at[pl.ds(base_c + s * shf, shf)])
      plsc.subcore_barrier()

    phase(s1, d1, r1, o1, 2, C_R)
    phase(s2, d2, r2, o2, 8, C_M)
    phase(s3, d3, r3, o3, 8, C_M)

  return k


def _make_seg2(n_src_pad, n_dst_pad, nchunk, C, counts_mode):
  npass = nchunk // 2
  rpt = C // NS
  n64 = rpt // 64
  rem = rpt - n64 * 64
  mesh = plsc.VectorSubcoreMesh(core_axis_name="c", subcore_axis_name="s",
                                num_cores=NC, num_subcores=NS)

  SH = 98

  @functools.partial(
      pl.kernel,
      out_type=jax.ShapeDtypeStruct((n_dst_pad, D), _f32),
      mesh=mesh,
      scratch_types=[
          pltpu.VMEM((7 * 2048,), _i32),
          pltpu.VMEM((SH, G), _i32),
          pltpu.VMEM((3, G, D), _f32),
          pltpu.VMEM((16, D), _f32),
          pltpu.VMEM((8, 128), _i32),
          pltpu.VMEM_SHARED((C + 8, D), _f32),
          pltpu.SemaphoreType.DMA,
          pltpu.SemaphoreType.DMA,
          pltpu.SemaphoreType.DMA,
          pltpu.SemaphoreType.DMA,
          pltpu.SemaphoreType.DMA,
          pltpu.SemaphoreType.DMA,
          pltpu.SemaphoreType.DMA,
          pltpu.SemaphoreType.DMA,
      ],
  )
  def k(table_hbm, packed_hbm, counts_hbm, out_hbm,
        csrc, cdst, rows_v, zb_v, cnt2, acc_sp, sem0, sem1,
        g0, g1, g2, t0, t1, t2):
    c = lax.axis_index("c")
    s = lax.axis_index("s")
    pltpu.sync_copy(counts_hbm, cnt2)

    def zb_body(i, _):
      r = i // 8
      col = (i % 8) * 16
      zb_v[r, pl.ds(col, 16)] = jnp.zeros((16,), _f32)
      return 0
    lax.fori_loop(0, 16 * 8, zb_body, 0)
    if counts_mode:
      def ob_body(i, _):
        r = i // 8
        col = (i % 8) * 16
        rows_v[0, r, pl.ds(col, 16)] = jnp.ones((16,), _f32)
        return 0
      lax.fori_loop(0, G * 8, ob_body, 0)

    for p in range(npass):
      kk = c * npass + p
      lo = kk * C

      nz = rpt // 16
      rz = rpt - nz * 16
      def z_body(i, _):
        off = pl.multiple_of(s * rpt + i * 16, 8)
        pltpu.async_copy(zb_v, acc_sp.at[pl.ds(off, 16)], sem0)
        return 0
      lax.fori_loop(0, nz, z_body, 0)
      if rz:
        off = pl.multiple_of(s * rpt + nz * 16, 8)
        pltpu.async_copy(zb_v.at[pl.ds(0, rz)], acc_sp.at[pl.ds(off, rz)],
                         sem0)
      def z_drain(i, _):
        off = pl.multiple_of(s * rpt + i * 16, 8)
        pltpu.make_async_copy(zb_v, acc_sp.at[pl.ds(off, 16)], sem0).wait()
        return 0
      lax.fori_loop(0, nz, z_drain, 0)
      if rz:
        off = pl.multiple_of(s * rpt + nz * 16, 8)
        pltpu.make_async_copy(zb_v.at[pl.ds(0, rz)],
                              acc_sp.at[pl.ds(off, rz)], sem0).wait()
      plsc.subcore_barrier()

      ck = cnt2[kk, pl.ds(0, 16)][0]
      nblk = lax.shift_right_logical(ck + (G - 1), 7)
      share = lax.shift_right_logical(nblk + (NS - 1), 4)

      wbase = kk * REG + s * share * G
      nst = lax.shift_right_logical(share + 15, 4)
      def st_body(i, _):
        off = pl.multiple_of(wbase + i * 2048, 8)
        pltpu.async_copy(packed_hbm.at[pl.ds(off, 2048)],
                         csrc.at[pl.ds(i * 2048, 2048)], sem1)
        return 0
      lax.fori_loop(0, nst, st_body, 0)
      def st_drain(i, _):
        off = pl.multiple_of(wbase + i * 2048, 8)
        pltpu.make_async_copy(packed_hbm.at[pl.ds(off, 2048)],
                              csrc.at[pl.ds(i * 2048, 2048)], sem1).wait()
        return 0
      lax.fori_loop(0, nst, st_drain, 0)

      def up_body(i, _):
        r = i // 8
        cu = (i % 8) * 16
        v = csrc[pl.ds(i * 16, 16)]
        if not counts_mode:
          csrc[pl.ds(i * 16, 16)] = lax.bitwise_and(v, PK - 1)
        cdst[r, pl.ds(cu, 16)] = lax.shift_right_logical(v, 17)
        return 0
      lax.fori_loop(0, share * 8, up_body, 0)

      gsem = (g0, g1, g2)
      tsem = (t0, t1, t2)
      def fireg(j, u):
        pltpu.async_copy(table_hbm.at[csrc.at[pl.ds(j * G, G)]],
                         rows_v.at[u], gsem[u])
      def waitg(j, u):
        pltpu.make_async_copy(table_hbm.at[csrc.at[pl.ds(j * G, G)]],
                              rows_v.at[u], gsem[u]).wait()
      def fires(j, u):
        pltpu.async_copy(rows_v.at[u], acc_sp.at[cdst.at[j]], tsem[u],
                         add=True)
      def waits(j, u):
        pltpu.make_async_copy(rows_v.at[u], acc_sp.at[cdst.at[j]],
                              tsem[u]).wait()

      if counts_mode:
        def cb(b, _):
          pltpu.sync_copy(rows_v.at[0], acc_sp.at[cdst.at[b]], add=True)
          return 0
        lax.fori_loop(0, share, cb, 0)
      else:
        def pip(tt, _):
          for u in range(3):
            j = tt * 3 + u
            um = (u + 2) % 3
            @pl.when((j >= 3) & (j - 3 < share))
            def _():
              waits(j - 3, u)
            @pl.when(j < share)
            def _():
              fireg(j, u)
            @pl.when((j >= 1) & (j - 1 < share))
            def _():
              waitg(j - 1, um)
              fires(j - 1, um)
          return 0
        lax.fori_loop(0, lax.div(share + 5, jnp.int32(3)), pip, 0)
      plsc.subcore_barrier()

      def w_body(i, _):
        aoff = pl.multiple_of(s * rpt + i * 64, 8)
        ooff = pl.multiple_of(lo + s * rpt + i * 64, 8)
        pltpu.async_copy(acc_sp.at[pl.ds(aoff, 64)],
                         out_hbm.at[pl.ds(ooff, 64)], sem0)
        return 0
      lax.fori_loop(0, n64, w_body, 0)
      if rem:
        aoff = pl.multiple_of(s * rpt + n64 * 64, 8)
        ooff = pl.multiple_of(lo + s * rpt + n64 * 64, 8)
        pltpu.async_copy(acc_sp.at[pl.ds(aoff, rem)],
                         out_hbm.at[pl.ds(ooff, rem)], sem0)
      def w_drain(i, _):
        aoff = pl.multiple_of(s * rpt + i * 64, 8)
        ooff = pl.multiple_of(lo + s * rpt + i * 64, 8)
        pltpu.make_async_copy(acc_sp.at[pl.ds(aoff, 64)],
                              out_hbm.at[pl.ds(ooff, 64)], sem0).wait()
        return 0
      lax.fori_loop(0, n64, w_drain, 0)
      if rem:
        aoff = pl.multiple_of(s * rpt + n64 * 64, 8)
        ooff = pl.multiple_of(lo + s * rpt + n64 * 64, 8)
        pltpu.make_async_copy(acc_sp.at[pl.ds(aoff, rem)],
                              out_hbm.at[pl.ds(ooff, rem)], sem0).wait()
      plsc.subcore_barrier()

  return k


def _make_segmm():
  mesh = plsc.VectorSubcoreMesh(core_axis_name="c", subcore_axis_name="s",
                                num_cores=NC, num_subcores=NS)

  SH = 98

  @functools.partial(
      pl.kernel,
      out_type=(jax.ShapeDtypeStruct((NM_PAD, D), _f32),
                jax.ShapeDtypeStruct((NM_PAD, D), _f32)),
      mesh=mesh,
      scratch_types=[
          pltpu.VMEM((7 * 2048,), _i32),
          pltpu.VMEM((SH, G), _i32),
          pltpu.VMEM((3, G, D), _f32),
          pltpu.VMEM((16, D), _f32),
          pltpu.VMEM((8, 128), _i32),
          pltpu.VMEM_SHARED((C_M + 8, D), _f32),
          pltpu.SemaphoreType.DMA,
          pltpu.SemaphoreType.DMA,
          pltpu.SemaphoreType.DMA,
          pltpu.SemaphoreType.DMA,
          pltpu.SemaphoreType.DMA,
          pltpu.SemaphoreType.DMA,
          pltpu.SemaphoreType.DMA,
          pltpu.SemaphoreType.DMA,
      ],
  )
  def k(xr_hbm, xm_hbm, p2_hbm, c2_hbm, p3_hbm, c3_hbm, o2_hbm, o3_hbm,
        csrc, cdst, rows_v, zb_v, cnt2, acc_sp, sem0, sem1,
        g0, g1, g2, t0, t1, t2):
    c = lax.axis_index("c")
    s = lax.axis_index("s")
    npass = 4
    C = C_M
    rpt = C // NS
    n64 = rpt // 64
    rem = rpt - n64 * 64

    def zb_body(i, _):
      r = i // 8
      col = (i % 8) * 16
      zb_v[r, pl.ds(col, 16)] = jnp.zeros((16,), _f32)
      return 0
    lax.fori_loop(0, 16 * 8, zb_body, 0)
    def one_agg(table_hbm, packed_hbm, counts_hbm, out_hbm):
     pltpu.sync_copy(counts_hbm, cnt2)
     for p in range(npass):
      kk = c * npass + p
      lo = kk * C

      nz = rpt // 16
      rz = rpt - nz * 16
      def z_body(i, _):
        off = pl.multiple_of(s * rpt + i * 16, 8)
        pltpu.async_copy(zb_v, acc_sp.at[pl.ds(off, 16)], sem0)
        return 0
      lax.fori_loop(0, nz, z_body, 0)
      if rz:
        off = pl.multiple_of(s * rpt + nz * 16, 8)
        pltpu.async_copy(zb_v.at[pl.ds(0, rz)], acc_sp.at[pl.ds(off, rz)],
                         sem0)
      def z_drain(i, _):
        off = pl.multiple_of(s * rpt + i * 16, 8)
        pltpu.make_async_copy(zb_v, acc_sp.at[pl.ds(off, 16)], sem0).wait()
        return 0
      lax.fori_loop(0, nz, z_drain, 0)
      if rz:
        off = pl.multiple_of(s * rpt + nz * 16, 8)
        pltpu.make_async_copy(zb_v.at[pl.ds(0, rz)],
                              acc_sp.at[pl.ds(off, rz)], sem0).wait()
      plsc.subcore_barrier()

      ck = cnt2[kk, pl.ds(0, 16)][0]
      nblk = lax.shift_right_logical(ck + (G - 1), 7)
      share = lax.shift_right_logical(nblk + (NS - 1), 4)

      wbase = kk * REG + s * share * G
      nst = lax.shift_right_logical(share + 15, 4)
      def st_body(i, _):
        off = pl.multiple_of(wbase + i * 2048, 8)
        pltpu.async_copy(packed_hbm.at[pl.ds(off, 2048)],
                         csrc.at[pl.ds(i * 2048, 2048)], sem1)
        return 0
      lax.fori_loop(0, nst, st_body, 0)
      def st_drain(i, _):
        off = pl.multiple_of(wbase + i * 2048, 8)
        pltpu.make_async_copy(packed_hbm.at[pl.ds(off, 2048)],
                              csrc.at[pl.ds(i * 2048, 2048)], sem1).wait()
        return 0
      lax.fori_loop(0, nst, st_drain, 0)

      def up_body(i, _):
        r = i // 8
        cu = (i % 8) * 16
        v = csrc[pl.ds(i * 16, 16)]
        csrc[pl.ds(i * 16, 16)] = lax.bitwise_and(v, PK - 1)
        cdst[r, pl.ds(cu, 16)] = lax.shift_right_logical(v, 17)
        return 0
      lax.fori_loop(0, share * 8, up_body, 0)

      gsem = (g0, g1, g2)
      tsem = (t0, t1, t2)
      def fireg(j, u):
        pltpu.async_copy(table_hbm.at[csrc.at[pl.ds(j * G, G)]],
                         rows_v.at[u], gsem[u])
      def waitg(j, u):
        pltpu.make_async_copy(table_hbm.at[csrc.at[pl.ds(j * G, G)]],
                              rows_v.at[u], gsem[u]).wait()
      def fires(j, u):
        pltpu.async_copy(rows_v.at[u], acc_sp.at[cdst.at[j]], tsem[u],
                         add=True)
      def waits(j, u):
        pltpu.make_async_copy(rows_v.at[u], acc_sp.at[cdst.at[j]],
                              tsem[u]).wait()

      def pip(tt, _):
        for u in range(3):
          j = tt * 3 + u
          um = (u + 2) % 3
          @pl.when((j >= 3) & (j - 3 < share))
          def _():
            waits(j - 3, u)
          @pl.when(j < share)
          def _():
            fireg(j, u)
          @pl.when((j >= 1) & (j - 1 < share))
          def _():
            waitg(j - 1, um)
            fires(j - 1, um)
        return 0
      lax.fori_loop(0, lax.div(share + 5, jnp.int32(3)), pip, 0)
      plsc.subcore_barrier()

      def w_body(i, _):
        aoff = pl.multiple_of(s * rpt + i * 64, 8)
        ooff = pl.multiple_of(lo + s * rpt + i * 64, 8)
        pltpu.async_copy(acc_sp.at[pl.ds(aoff, 64)],
                         out_hbm.at[pl.ds(ooff, 64)], sem0)
        return 0
      lax.fori_loop(0, n64, w_body, 0)
      if rem:
        aoff = pl.multiple_of(s * rpt + n64 * 64, 8)
        ooff = pl.multiple_of(lo + s * rpt + n64 * 64, 8)
        pltpu.async_copy(acc_sp.at[pl.ds(aoff, rem)],
                         out_hbm.at[pl.ds(ooff, rem)], sem0)
      def w_drain(i, _):
        aoff = pl.multiple_of(s * rpt + i * 64, 8)
        ooff = pl.multiple_of(lo + s * rpt + i * 64, 8)
        pltpu.make_async_copy(acc_sp.at[pl.ds(aoff, 64)],
                              out_hbm.at[pl.ds(ooff, 64)], sem0).wait()
        return 0
      lax.fori_loop(0, n64, w_drain, 0)
      if rem:
        aoff = pl.multiple_of(s * rpt + n64 * 64, 8)
        ooff = pl.multiple_of(lo + s * rpt + n64 * 64, 8)
        pltpu.make_async_copy(acc_sp.at[pl.ds(aoff, rem)],
                              out_hbm.at[pl.ds(ooff, rem)], sem0).wait()
      plsc.subcore_barrier()

    one_agg(xr_hbm, p2_hbm, c2_hbm, o2_hbm)
    one_agg(xm_hbm, p3_hbm, c3_hbm, o3_hbm)

  return k


def _make_cnt(n_dst_pad, nchunk, C):
  npass = nchunk // 2
  rpt = C // NS
  SH = 98
  mesh = plsc.VectorSubcoreMesh(core_axis_name="c", subcore_axis_name="s",
                                num_cores=NC, num_subcores=NS)

  @functools.partial(
      pl.kernel,
      out_type=jax.ShapeDtypeStruct((n_dst_pad,), _f32),
      mesh=mesh,
      scratch_types=[
          pltpu.VMEM((7 * 2048,), _i32),
          pltpu.VMEM((SH, G), _i32),
          pltpu.VMEM((G,), _f32),
          pltpu.VMEM((2048,), _f32),
          pltpu.VMEM((2048,), _f32),
          pltpu.VMEM((8, 128), _i32),
          pltpu.VMEM_SHARED((C + 8,), _f32),
          pltpu.SemaphoreType.DMA,
          pltpu.SemaphoreType.DMA,
      ],
  )
  def k(packed_hbm, counts_hbm, out_hbm,
        pk_v, cdst, ones_v, zb_v, wb_v, cnt2, acc_sp, sem0, sem1):
    c = lax.axis_index("c")
    s = lax.axis_index("s")
    pltpu.sync_copy(counts_hbm, cnt2)
    def zi(i, _):
      zb_v[pl.ds(i * 16, 16)] = jnp.zeros((16,), _f32)
      return 0
    lax.fori_loop(0, 2048 // 16, zi, 0)
    def oi(i, _):
      ones_v[pl.ds(i * 16, 16)] = jnp.ones((16,), _f32)
      return 0
    lax.fori_loop(0, G // 16, oi, 0)

    nw = NS
    wrt = C // nw
    while wrt % 16:
      nw //= 2
      wrt = C // nw

    for p in range(npass):
      kk = c * npass + p
      lo = kk * C
      off0 = pl.multiple_of(s * wrt, 16)
      @pl.when(s < nw)
      def _():
        pltpu.sync_copy(zb_v.at[pl.ds(0, wrt)], acc_sp.at[pl.ds(off0, wrt)])
      plsc.subcore_barrier()

      ck = cnt2[kk, pl.ds(0, 16)][0]
      nblk = lax.shift_right_logical(ck + (G - 1), 7)
      share = lax.shift_right_logical(nblk + (NS - 1), 4)
      wbase = kk * REG + s * share * G
      nst = lax.shift_right_logical(share + 15, 4)
      def st_body(i, _):
        off = pl.multiple_of(wbase + i * 2048, 8)
        pltpu.async_copy(packed_hbm.at[pl.ds(off, 2048)],
                         pk_v.at[pl.ds(i * 2048, 2048)], sem1)
        return 0
      lax.fori_loop(0, nst, st_body, 0)
      def st_drain(i, _):
        off = pl.multiple_of(wbase + i * 2048, 8)
        pltpu.make_async_copy(packed_hbm.at[pl.ds(off, 2048)],
                              pk_v.at[pl.ds(i * 2048, 2048)], sem1).wait()
        return 0
      lax.fori_loop(0, nst, st_drain, 0)
      def up_body(i, _):
        r = i // 8
        cu = (i % 8) * 16
        v = pk_v[pl.ds(i * 16, 16)]
        cdst[r, pl.ds(cu, 16)] = lax.shift_right_logical(v, 17)
        return 0
      lax.fori_loop(0, share * 8, up_body, 0)
      def cb(b, _):
        pltpu.sync_copy(ones_v, acc_sp.at[cdst.at[b]], add=True)
        return 0
      lax.fori_loop(0, share, cb, 0)
      plsc.subcore_barrier()
      ooff = pl.multiple_of(lo + s * wrt, 16)
      @pl.when(s < nw)
      def _():
        pltpu.sync_copy(acc_sp.at[pl.ds(off0, wrt)], wb_v.at[pl.ds(0, wrt)])
        pltpu.sync_copy(wb_v.at[pl.ds(0, wrt)], out_hbm.at[pl.ds(ooff, wrt)])
      plsc.subcore_barrier()

  return k


_sc_cache = {}


def _sc_kernels():
  if "k" not in _sc_cache:
    _sc_cache["k"] = (
        _make_rank(2, C_R),
        _make_rank(8, C_M),
        _make_permute(),
        _make_seg2(NM_PAD, NR_PAD, 2, C_R, False),
        _make_segmm(),
        _make_cnt(NR_PAD, 2, C_R),
        _make_cnt(NM_PAD, 8, C_M),
    )
  return _sc_cache["k"]


_BLK = 1024


def _dense_r_body(s_ref, cnt_ref, x_ref, wn_ref, wr_ref, b_ref, o_ref):
  agg = s_ref[...] / jnp.maximum(cnt_ref[...], 1.0)
  o_ref[...] = (jnp.dot(agg, wn_ref[...], preferred_element_type=_f32)
                + jnp.dot(x_ref[...], wr_ref[...], preferred_element_type=_f32)
                + b_ref[0:1, :])


def _dense_r(n_pad, s, cnt, x, wn, wr, b):
  grid = (n_pad // _BLK,)
  return pl.pallas_call(
      _dense_r_body,
      grid=grid,
      in_specs=[
          pl.BlockSpec((_BLK, D), lambda i: (i, 0)),
          pl.BlockSpec((_BLK, 1), lambda i: (i, 0)),
          pl.BlockSpec((_BLK, D), lambda i: (i, 0)),
          pl.BlockSpec((D, D), lambda i: (0, 0)),
          pl.BlockSpec((D, D), lambda i: (0, 0)),
          pl.BlockSpec((8, D), lambda i: (0, 0)),
      ],
      out_specs=pl.BlockSpec((_BLK, D), lambda i: (i, 0)),
      out_shape=jax.ShapeDtypeStruct((n_pad, D), _f32),
  )(s, cnt, x, wn, wr, b)


def _dense_m_body(s1_ref, c1_ref, s2_ref, c2_ref, x_ref,
                  wn1_ref, wr1_ref, b1_ref, wn2_ref, wr2_ref, b2_ref, o_ref):
  a1 = s1_ref[...] / jnp.maximum(c1_ref[...], 1.0)
  a2 = s2_ref[...] / jnp.maximum(c2_ref[...], 1.0)
  x = x_ref[...]
  o_ref[...] = (jnp.dot(a1, wn1_ref[...], preferred_element_type=_f32)
                + jnp.dot(x, wr1_ref[...] + wr2_ref[...],
                          preferred_element_type=_f32)
                + jnp.dot(a2, wn2_ref[...], preferred_element_type=_f32)
                + (b1_ref[0:1, :] + b2_ref[0:1, :]))


def _dense_m(n_pad, s1, c1, s2, c2, x, wn1, wr1, b1, wn2, wr2, b2):
  grid = (n_pad // _BLK,)
  row = lambda i: (i, 0)
  full = lambda i: (0, 0)
  return pl.pallas_call(
      _dense_m_body,
      grid=grid,
      in_specs=[
          pl.BlockSpec((_BLK, D), row),
          pl.BlockSpec((_BLK, 1), row),
          pl.BlockSpec((_BLK, D), row),
          pl.BlockSpec((_BLK, 1), row),
          pl.BlockSpec((_BLK, D), row),
          pl.BlockSpec((D, D), full),
          pl.BlockSpec((D, D), full),
          pl.BlockSpec((8, D), full),
          pl.BlockSpec((D, D), full),
          pl.BlockSpec((D, D), full),
          pl.BlockSpec((8, D), full),
      ],
      out_specs=pl.BlockSpec((_BLK, D), row),
      out_shape=jax.ShapeDtypeStruct((n_pad, D), _f32),
  )(s1, c1, s2, c2, x, wn1, wr1, b1, wn2, wr2, b2)


def _head_body(x_ref, emb_ref, ty_ref, wlr_ref, blr_ref,
               w0_ref, w1_ref, bt_ref, o_ref):
  ro = jnp.dot(x_ref[...], wlr_ref[...], preferred_element_type=_f32) \
      + blr_ref[0:1, :]
  emb = emb_ref[...]
  o0 = jnp.dot(emb, w0_ref[...], preferred_element_type=_f32) + bt_ref[0:1, :]
  o1 = jnp.dot(emb, w1_ref[...], preferred_element_type=_f32) + bt_ref[1:2, :]
  sel = jnp.where(ty_ref[...] == 0, o0, o1)
  dot = jnp.sum(ro * sel, axis=1)
  na = jnp.sqrt(jnp.sum(ro * ro, axis=1))
  nb = jnp.sqrt(jnp.sum(sel * sel, axis=1))
  cos = dot / (jnp.maximum(na, 1e-8) * jnp.maximum(nb, 1e-8))
  o_ref[...] = (cos + 1.0) * 0.5


def _head(x, emb, ty, wlr, blr, w0, w1, bt):
  grid = (NR_PAD // _BLK,)
  row = lambda i: (i, 0)
  full = lambda i: (0, 0)
  return pl.pallas_call(
      _head_body,
      grid=grid,
      in_specs=[
          pl.BlockSpec((_BLK, D), row),
          pl.BlockSpec((_BLK, EMB), row),
          pl.BlockSpec((_BLK, 1), row),
          pl.BlockSpec((D, D), full),
          pl.BlockSpec((8, D), full),
          pl.BlockSpec((EMB, D), full),
          pl.BlockSpec((EMB, D), full),
          pl.BlockSpec((8, D), full),
      ],
      out_specs=pl.BlockSpec((_BLK,), lambda i: (i,)),
      out_shape=jax.ShapeDtypeStruct((NR_PAD,), _f32),
  )(x, emb, ty, wlr, blr, w0, w1, bt)


def _pad_bias(b):
  return jnp.pad(b.reshape(1, D), ((0, 7), (0, 0)))


def kernel(x_reaction, x_molecule, ei_m2r, ei_r2m, ei_m2m, output_notes_opt,
           output_nodes_types, Wn, Wr, bconv, W_lr, b_lr, W_types, b_types):
  x_r = jnp.pad(x_reaction, ((0, NR_PAD - N_R), (0, 0)))
  x_m = jnp.pad(x_molecule, ((0, NM_PAD - N_M), (0, 0)))

  def split(ei):
    src = jnp.pad(ei[0].astype(_i32), (0, E_PAD - E))
    dst = jnp.pad(ei[1].astype(_i32), (0, E_PAD - E), constant_values=-1)
    return src, dst

  (_rank_r, _rank_m, _permute,
   _seg_m2r, _seg_mm, _cntk_r, _cntk_m) = _sc_kernels()

  s_m2r, d_m2r = split(ei_m2r)
  s_r2m, d_r2m = split(ei_r2m)
  s_m2m, d_m2m = split(ei_m2m)

  rk1, ct1 = _rank_r(d_m2r.reshape(EBLK, 8, 128))
  rk2, ct2 = _rank_m(d_r2m.reshape(EBLK, 8, 128))
  rk3, ct3 = _rank_m(d_m2m.reshape(EBLK, 8, 128))

  p1, p2, p3 = _permute(
      s_m2r, d_m2r, rk1.reshape(NS, ET // G, G),
      s_r2m, d_r2m, rk2.reshape(NS, ET // G, G),
      s_m2m, d_m2m, rk3.reshape(NS, ET // G, G))

  cnt_r = _cntk_r(p1, ct1).reshape(NR_PAD, 1)
  cnt_m1 = _cntk_m(p2, ct2).reshape(NM_PAD, 1)
  cnt_m2 = _cntk_m(p3, ct3).reshape(NM_PAD, 1)

  for l in range(L):
    agg_r = _seg_m2r(x_m, p1, ct1)
    agg_m1, agg_m2 = _seg_mm(x_r, x_m, p2, ct2, p3, ct3)
    x_r_new = _dense_r(NR_PAD, agg_r, cnt_r, x_r,
                       Wn[l, 0], Wr[l, 0], _pad_bias(bconv[l, 0]))
    x_m_new = _dense_m(NM_PAD, agg_m1, cnt_m1, agg_m2, cnt_m2, x_m,
                       Wn[l, 1], Wr[l, 1], _pad_bias(bconv[l, 1]),
                       Wn[l, 2], Wr[l, 2], _pad_bias(bconv[l, 2]))
    x_r, x_m = x_r_new, x_m_new

  emb = jnp.pad(output_notes_opt, ((0, NR_PAD - N_R), (0, 0)))
  ty = jnp.pad(output_nodes_types.astype(_i32),
               (0, NR_PAD - N_R)).reshape(NR_PAD, 1)
  bt = jnp.pad(b_types, ((0, 6), (0, 0)))
  out = _head(x_r, emb, ty, W_lr, _pad_bias(b_lr),
              W_types[0], W_types[1], bt)
  return out[:N_R]


# speedup vs baseline: 1.1922x; 1.0023x over previous
import functools

import jax
import jax.numpy as jnp
from jax import lax
from jax.experimental import pallas as pl
from jax.experimental.pallas import tpu as pltpu
from jax.experimental.pallas import tpu_sc as plsc

N_R = 10000
N_M = 50000
D = 128
EMB = 1024
L = 4
E = 200000

NR_PAD = 10240
NM_PAD = 50176
E_PAD = 200704
EBLK = E_PAD // 1024

NC = 2
NS = 16
ET = E_PAD // NS
G = 128
REG = 204800
PK = 131072

C_R = NR_PAD // 2
C_M = NM_PAD // 8

_f32 = jnp.float32
_i32 = jnp.int32


def _prefix_8x128(m):
  li = lax.broadcasted_iota(_i32, (8, 128), 1)
  x = m
  for k in (1, 2, 4, 8, 16, 32, 64):
    x = x + jnp.where(li >= k, pltpu.roll(x, k, 1), 0)
  rt = x[:, 127:128]
  si = lax.broadcasted_iota(_i32, (8, 1), 0)
  r = rt
  for k in (1, 2, 4):
    r = r + jnp.where(si >= k, pltpu.roll(r, k, 0), 0)
  return x + (r - rt)


def _make_rank(nchunk, C):
  nreg = nchunk + 1

  def body(d_ref, rank_ref, cnt_ref, carry):
    pid = pl.program_id(0)

    @pl.when(pid == 0)
    def _():
      for k in range(nreg):
        carry[k] = 0

    d = d_ref[0]
    rank = jnp.zeros((8, 128), _i32)
    for k in range(nreg):
      if k < nchunk:
        mk = (d >= k * C) & (d < (k + 1) * C)
      else:
        mk = d < 0
      mi = jnp.where(mk, 1, 0).astype(_i32)
      incl = _prefix_8x128(mi)
      ck = carry[k]
      rank = rank + jnp.where(mk, k * REG + ck + (incl - mi), 0)
      carry[k] = ck + jnp.sum(mi)
    rank_ref[0] = rank

    si = lax.broadcasted_iota(_i32, (8, 128), 0)
    li = lax.broadcasted_iota(_i32, (8, 128), 1)
    cvals = jnp.zeros((8, 128), _i32)
    for k in range(nreg):
      cvals = cvals + jnp.where((si == k) & (li == 0), carry[k], 0)
    cnt_ref[...] = cvals

  def call(d3):
    return pl.pallas_call(
        body,
        grid=(EBLK,),
        in_specs=[pl.BlockSpec((1, 8, 128), lambda i: (i, 0, 0))],
        out_specs=[pl.BlockSpec((1, 8, 128), lambda i: (i, 0, 0)),
                   pl.BlockSpec((8, 128), lambda i: (0, 0))],
        out_shape=[jax.ShapeDtypeStruct((EBLK, 8, 128), _i32),
                   jax.ShapeDtypeStruct((8, 128), _i32)],
        scratch_shapes=[pltpu.SMEM((8,), _i32)],
    )(d3)

  return call


def _make_permute():
  mesh = plsc.VectorSubcoreMesh(core_axis_name="c", subcore_axis_name="s",
                                num_cores=NC, num_subcores=NS)
  nrow = ET // G

  PB = 2048

  @functools.partial(
      pl.kernel,
      out_type=(jax.ShapeDtypeStruct((2 * REG,), _i32),
                jax.ShapeDtypeStruct((8 * REG,), _i32),
                jax.ShapeDtypeStruct((8 * REG,), _i32)),
      mesh=mesh,
      scratch_types=[
          pltpu.VMEM((ET,), _i32),
          pltpu.VMEM((ET,), _i32),
          pltpu.VMEM((nrow, G), _i32),
          pltpu.VMEM((nrow, G), _i32),
          pltpu.VMEM((nrow, G), _i32),
          pltpu.VMEM((PB,), _i32),
          pltpu.VMEM_SHARED((4 * REG + PB,), _i32),
          pltpu.SemaphoreType.DMA,
      ],
  )
  def k(s1, d1, r1, s2, d2, r2, s3, d3, r3, o1, o2, o3,
        src_v, dst_v, rank2, val2, sidx, padb, spbuf, sem):
    c = lax.axis_index("c")
    s = lax.axis_index("s")
    iota16 = lax.iota(_i32, 16)

    def phase(s_hbm, d_hbm, r3_hbm, out_hbm, nchunk, C):
      npc = nchunk // NC
      own = npc * REG
      base_c = c * own
      padval = jnp.full((16,), C * PK, _i32)

      def pfill(i, _):
        padb[pl.ds(i * 16, 16)] = padval
        return 0
      lax.fori_loop(0, PB // 16, pfill, 0)
      shf = own // NS
      def sfill(f, _):
        pltpu.async_copy(padb, spbuf.at[pl.ds(s * shf + f * PB, PB)], sem)
        return 0
      lax.fori_loop(0, shf // PB, sfill, 0)
      def sfill_d(f, _):
        pltpu.make_async_copy(padb, spbuf.at[pl.ds(s * shf + f * PB, PB)],
                              sem).wait()
        return 0
      lax.fori_loop(0, shf // PB, sfill_d, 0)
      if shf % PB:
        pltpu.sync_copy(padb.at[pl.ds(0, shf % PB)],
                        spbuf.at[pl.ds(s * shf + (shf // PB) * PB,
                                       shf % PB)])

      pltpu.sync_copy(s_hbm.at[pl.ds(s * ET, ET)], src_v)
      pltpu.sync_copy(d_hbm.at[pl.ds(s * ET, ET)], dst_v)
      pltpu.sync_copy(r3_hbm.at[s], rank2)

      def pack(i, _):
        r = i // 8
        cc = (i % 8) * 16
        sv = src_v[pl.ds(i * 16, 16)]
        dv = dst_v[pl.ds(i * 16, 16)]
        rk = rank2[r, pl.ds(cc, 16)]
        kk = jnp.zeros((16,), _i32)
        for q in range(1, nchunk):
          kk = kk + jnp.where(dv >= q * C, 1, 0)
        doff = jnp.where(dv >= 0, dv - kk * C, C)
        val2[r, pl.ds(cc, 16)] = sv + doff * PK
        mine = (rk >= base_c) & (rk < base_c + own)
        tr = 4 * REG + lax.bitwise_and(i * 16 + iota16, PB - 1)
        sidx[r, pl.ds(cc, 16)] = jnp.where(mine, rk - base_c, tr)
        return 0
      lax.fori_loop(0, ET // 16, pack, 0)
      plsc.subcore_barrier()

      def scat(j, _):
        pltpu.async_copy(val2.at[j], spbuf.at[sidx.at[j]], sem)
        return 0
      lax.fori_loop(0, nrow, scat, 0)
      def drain(j, _):
        pltpu.make_async_copy(val2.at[j], spbuf.at[sidx.at[j]], sem).wait()
        return 0
      lax.fori_loop(0, nrow, drain, 0)
      plsc.subcore_barrier()

      pltpu.sync_copy(spbuf.at[pl.ds(s * shf, shf)],
                      out_hbm.at[pl.ds(base_c + s * shf, shf)])
      plsc.subcore_barrier()

    phase(s1, d1, r1, o1, 2, C_R)
    phase(s2, d2, r2, o2, 8, C_M)
    phase(s3, d3, r3, o3, 8, C_M)

  return k


def _make_seg2(n_src_pad, n_dst_pad, nchunk, C, counts_mode):
  npass = nchunk // 2
  rpt = C // NS
  n64 = rpt // 64
  rem = rpt - n64 * 64
  mesh = plsc.VectorSubcoreMesh(core_axis_name="c", subcore_axis_name="s",
                                num_cores=NC, num_subcores=NS)

  SH = 98

  @functools.partial(
      pl.kernel,
      out_type=jax.ShapeDtypeStruct((n_dst_pad, D), _f32),
      mesh=mesh,
      scratch_types=[
          pltpu.VMEM((7 * 2048,), _i32),
          pltpu.VMEM((SH, G), _i32),
          pltpu.VMEM((3, G, D), _f32),
          pltpu.VMEM((16, D), _f32),
          pltpu.VMEM((8, 128), _i32),
          pltpu.VMEM_SHARED((C + 8, D), _f32),
          pltpu.SemaphoreType.DMA,
          pltpu.SemaphoreType.DMA,
          pltpu.SemaphoreType.DMA,
          pltpu.SemaphoreType.DMA,
          pltpu.SemaphoreType.DMA,
          pltpu.SemaphoreType.DMA,
          pltpu.SemaphoreType.DMA,
          pltpu.SemaphoreType.DMA,
      ],
  )
  def k(table_hbm, packed_hbm, counts_hbm, out_hbm,
        csrc, cdst, rows_v, zb_v, cnt2, acc_sp, sem0, sem1,
        g0, g1, g2, t0, t1, t2):
    c = lax.axis_index("c")
    s = lax.axis_index("s")
    pltpu.sync_copy(counts_hbm, cnt2)

    def zb_body(i, _):
      r = i // 8
      col = (i % 8) * 16
      zb_v[r, pl.ds(col, 16)] = jnp.zeros((16,), _f32)
      return 0
    lax.fori_loop(0, 16 * 8, zb_body, 0)
    if counts_mode:
      def ob_body(i, _):
        r = i // 8
        col = (i % 8) * 16
        rows_v[0, r, pl.ds(col, 16)] = jnp.ones((16,), _f32)
        return 0
      lax.fori_loop(0, G * 8, ob_body, 0)

    for p in range(npass):
      kk = c * npass + p
      lo = kk * C

      nz = rpt // 16
      rz = rpt - nz * 16
      def z_body(i, _):
        off = pl.multiple_of(s * rpt + i * 16, 8)
        pltpu.async_copy(zb_v, acc_sp.at[pl.ds(off, 16)], sem0)
        return 0
      lax.fori_loop(0, nz, z_body, 0)
      if rz:
        off = pl.multiple_of(s * rpt + nz * 16, 8)
        pltpu.async_copy(zb_v.at[pl.ds(0, rz)], acc_sp.at[pl.ds(off, rz)],
                         sem0)
      def z_drain(i, _):
        off = pl.multiple_of(s * rpt + i * 16, 8)
        pltpu.make_async_copy(zb_v, acc_sp.at[pl.ds(off, 16)], sem0).wait()
        return 0
      lax.fori_loop(0, nz, z_drain, 0)
      if rz:
        off = pl.multiple_of(s * rpt + nz * 16, 8)
        pltpu.make_async_copy(zb_v.at[pl.ds(0, rz)],
                              acc_sp.at[pl.ds(off, rz)], sem0).wait()
      plsc.subcore_barrier()

      ck = cnt2[kk, pl.ds(0, 16)][0]
      nblk = lax.shift_right_logical(ck + (G - 1), 7)
      share = lax.shift_right_logical(nblk + (NS - 1), 4)

      wbase = kk * REG + s * share * G
      nst = lax.shift_right_logical(share + 15, 4)
      def st_body(i, _):
        off = pl.multiple_of(wbase + i * 2048, 8)
        pltpu.async_copy(packed_hbm.at[pl.ds(off, 2048)],
                         csrc.at[pl.ds(i * 2048, 2048)], sem1)
        return 0
      lax.fori_loop(0, nst, st_body, 0)
      def st_drain(i, _):
        off = pl.multiple_of(wbase + i * 2048, 8)
        pltpu.make_async_copy(packed_hbm.at[pl.ds(off, 2048)],
                              csrc.at[pl.ds(i * 2048, 2048)], sem1).wait()
        return 0
      lax.fori_loop(0, nst, st_drain, 0)

      def up_body(i, _):
        r = i // 8
        cu = (i % 8) * 16
        v = csrc[pl.ds(i * 16, 16)]
        if not counts_mode:
          csrc[pl.ds(i * 16, 16)] = lax.bitwise_and(v, PK - 1)
        cdst[r, pl.ds(cu, 16)] = lax.shift_right_logical(v, 17)
        return 0
      lax.fori_loop(0, share * 8, up_body, 0)

      gsem = (g0, g1, g2)
      tsem = (t0, t1, t2)
      def fireg(j, u):
        pltpu.async_copy(table_hbm.at[csrc.at[pl.ds(j * G, G)]],
                         rows_v.at[u], gsem[u])
      def waitg(j, u):
        pltpu.make_async_copy(table_hbm.at[csrc.at[pl.ds(j * G, G)]],
                              rows_v.at[u], gsem[u]).wait()
      def fires(j, u):
        pltpu.async_copy(rows_v.at[u], acc_sp.at[cdst.at[j]], tsem[u],
                         add=True)
      def waits(j, u):
        pltpu.make_async_copy(rows_v.at[u], acc_sp.at[cdst.at[j]],
                              tsem[u]).wait()

      if counts_mode:
        def cb(b, _):
          pltpu.sync_copy(rows_v.at[0], acc_sp.at[cdst.at[b]], add=True)
          return 0
        lax.fori_loop(0, share, cb, 0)
      else:
        def pip(tt, _):
          for u in range(3):
            j = tt * 3 + u
            um = (u + 2) % 3
            @pl.when((j >= 3) & (j - 3 < share))
            def _():
              waits(j - 3, u)
            @pl.when(j < share)
            def _():
              fireg(j, u)
            @pl.when((j >= 1) & (j - 1 < share))
            def _():
              waitg(j - 1, um)
              fires(j - 1, um)
          return 0
        lax.fori_loop(0, lax.div(share + 5, jnp.int32(3)), pip, 0)
      plsc.subcore_barrier()

      def w_body(i, _):
        aoff = pl.multiple_of(s * rpt + i * 64, 8)
        ooff = pl.multiple_of(lo + s * rpt + i * 64, 8)
        pltpu.async_copy(acc_sp.at[pl.ds(aoff, 64)],
                         out_hbm.at[pl.ds(ooff, 64)], sem0)
        return 0
      lax.fori_loop(0, n64, w_body, 0)
      if rem:
        aoff = pl.multiple_of(s * rpt + n64 * 64, 8)
        ooff = pl.multiple_of(lo + s * rpt + n64 * 64, 8)
        pltpu.async_copy(acc_sp.at[pl.ds(aoff, rem)],
                         out_hbm.at[pl.ds(ooff, rem)], sem0)
      def w_drain(i, _):
        aoff = pl.multiple_of(s * rpt + i * 64, 8)
        ooff = pl.multiple_of(lo + s * rpt + i * 64, 8)
        pltpu.make_async_copy(acc_sp.at[pl.ds(aoff, 64)],
                              out_hbm.at[pl.ds(ooff, 64)], sem0).wait()
        return 0
      lax.fori_loop(0, n64, w_drain, 0)
      if rem:
        aoff = pl.multiple_of(s * rpt + n64 * 64, 8)
        ooff = pl.multiple_of(lo + s * rpt + n64 * 64, 8)
        pltpu.make_async_copy(acc_sp.at[pl.ds(aoff, rem)],
                              out_hbm.at[pl.ds(ooff, rem)], sem0).wait()
      plsc.subcore_barrier()

  return k


def _make_segmm():
  mesh = plsc.VectorSubcoreMesh(core_axis_name="c", subcore_axis_name="s",
                                num_cores=NC, num_subcores=NS)

  SH = 98

  @functools.partial(
      pl.kernel,
      out_type=(jax.ShapeDtypeStruct((NM_PAD, D), _f32),
                jax.ShapeDtypeStruct((NM_PAD, D), _f32)),
      mesh=mesh,
      scratch_types=[
          pltpu.VMEM((7 * 2048,), _i32),
          pltpu.VMEM((SH, G), _i32),
          pltpu.VMEM((3, G, D), _f32),
          pltpu.VMEM((16, D), _f32),
          pltpu.VMEM((8, 128), _i32),
          pltpu.VMEM_SHARED((C_M + 8, D), _f32),
          pltpu.SemaphoreType.DMA,
          pltpu.SemaphoreType.DMA,
          pltpu.SemaphoreType.DMA,
          pltpu.SemaphoreType.DMA,
          pltpu.SemaphoreType.DMA,
          pltpu.SemaphoreType.DMA,
          pltpu.SemaphoreType.DMA,
          pltpu.SemaphoreType.DMA,
      ],
  )
  def k(xr_hbm, xm_hbm, p2_hbm, c2_hbm, p3_hbm, c3_hbm, o2_hbm, o3_hbm,
        csrc, cdst, rows_v, zb_v, cnt2, acc_sp, sem0, sem1,
        g0, g1, g2, t0, t1, t2):
    c = lax.axis_index("c")
    s = lax.axis_index("s")
    npass = 4
    C = C_M
    rpt = C // NS
    n64 = rpt // 64
    rem = rpt - n64 * 64

    def zb_body(i, _):
      r = i // 8
      col = (i % 8) * 16
      zb_v[r, pl.ds(col, 16)] = jnp.zeros((16,), _f32)
      return 0
    lax.fori_loop(0, 16 * 8, zb_body, 0)
    def one_agg(table_hbm, packed_hbm, counts_hbm, out_hbm):
     pltpu.sync_copy(counts_hbm, cnt2)
     for p in range(npass):
      kk = c * npass + p
      lo = kk * C

      nz = rpt // 16
      rz = rpt - nz * 16
      def z_body(i, _):
        off = pl.multiple_of(s * rpt + i * 16, 8)
        pltpu.async_copy(zb_v, acc_sp.at[pl.ds(off, 16)], sem0)
        return 0
      lax.fori_loop(0, nz, z_body, 0)
      if rz:
        off = pl.multiple_of(s * rpt + nz * 16, 8)
        pltpu.async_copy(zb_v.at[pl.ds(0, rz)], acc_sp.at[pl.ds(off, rz)],
                         sem0)
      def z_drain(i, _):
        off = pl.multiple_of(s * rpt + i * 16, 8)
        pltpu.make_async_copy(zb_v, acc_sp.at[pl.ds(off, 16)], sem0).wait()
        return 0
      lax.fori_loop(0, nz, z_drain, 0)
      if rz:
        off = pl.multiple_of(s * rpt + nz * 16, 8)
        pltpu.make_async_copy(zb_v.at[pl.ds(0, rz)],
                              acc_sp.at[pl.ds(off, rz)], sem0).wait()
      plsc.subcore_barrier()

      ck = cnt2[kk, pl.ds(0, 16)][0]
      nblk = lax.shift_right_logical(ck + (G - 1), 7)
      share = lax.shift_right_logical(nblk + (NS - 1), 4)

      wbase = kk * REG + s * share * G
      nst = lax.shift_right_logical(share + 15, 4)
      def st_body(i, _):
        off = pl.multiple_of(wbase + i * 2048, 8)
        pltpu.async_copy(packed_hbm.at[pl.ds(off, 2048)],
                         csrc.at[pl.ds(i * 2048, 2048)], sem1)
        return 0
      lax.fori_loop(0, nst, st_body, 0)
      def st_drain(i, _):
        off = pl.multiple_of(wbase + i * 2048, 8)
        pltpu.make_async_copy(packed_hbm.at[pl.ds(off, 2048)],
                              csrc.at[pl.ds(i * 2048, 2048)], sem1).wait()
        return 0
      lax.fori_loop(0, nst, st_drain, 0)

      def up_body(i, _):
        r = i // 8
        cu = (i % 8) * 16
        v = csrc[pl.ds(i * 16, 16)]
        csrc[pl.ds(i * 16, 16)] = lax.bitwise_and(v, PK - 1)
        cdst[r, pl.ds(cu, 16)] = lax.shift_right_logical(v, 17)
        return 0
      lax.fori_loop(0, share * 8, up_body, 0)

      gsem = (g0, g1, g2)
      tsem = (t0, t1, t2)
      def fireg(j, u):
        pltpu.async_copy(table_hbm.at[csrc.at[pl.ds(j * G, G)]],
                         rows_v.at[u], gsem[u])
      def waitg(j, u):
        pltpu.make_async_copy(table_hbm.at[csrc.at[pl.ds(j * G, G)]],
                              rows_v.at[u], gsem[u]).wait()
      def fires(j, u):
        pltpu.async_copy(rows_v.at[u], acc_sp.at[cdst.at[j]], tsem[u],
                         add=True)
      def waits(j, u):
        pltpu.make_async_copy(rows_v.at[u], acc_sp.at[cdst.at[j]],
                              tsem[u]).wait()

      def pip(tt, _):
        for u in range(3):
          j = tt * 3 + u
          um = (u + 2) % 3
          @pl.when((j >= 3) & (j - 3 < share))
          def _():
            waits(j - 3, u)
          @pl.when(j < share)
          def _():
            fireg(j, u)
          @pl.when((j >= 1) & (j - 1 < share))
          def _():
            waitg(j - 1, um)
            fires(j - 1, um)
        return 0
      lax.fori_loop(0, lax.div(share + 5, jnp.int32(3)), pip, 0)
      plsc.subcore_barrier()

      def w_body(i, _):
        aoff = pl.multiple_of(s * rpt + i * 64, 8)
        ooff = pl.multiple_of(lo + s * rpt + i * 64, 8)
        pltpu.async_copy(acc_sp.at[pl.ds(aoff, 64)],
                         out_hbm.at[pl.ds(ooff, 64)], sem0)
        return 0
      lax.fori_loop(0, n64, w_body, 0)
      if rem:
        aoff = pl.multiple_of(s * rpt + n64 * 64, 8)
        ooff = pl.multiple_of(lo + s * rpt + n64 * 64, 8)
        pltpu.async_copy(acc_sp.at[pl.ds(aoff, rem)],
                         out_hbm.at[pl.ds(ooff, rem)], sem0)
      def w_drain(i, _):
        aoff = pl.multiple_of(s * rpt + i * 64, 8)
        ooff = pl.multiple_of(lo + s * rpt + i * 64, 8)
        pltpu.make_async_copy(acc_sp.at[pl.ds(aoff, 64)],
                              out_hbm.at[pl.ds(ooff, 64)], sem0).wait()
        return 0
      lax.fori_loop(0, n64, w_drain, 0)
      if rem:
        aoff = pl.multiple_of(s * rpt + n64 * 64, 8)
        ooff = pl.multiple_of(lo + s * rpt + n64 * 64, 8)
        pltpu.make_async_copy(acc_sp.at[pl.ds(aoff, rem)],
                              out_hbm.at[pl.ds(ooff, rem)], sem0).wait()
      plsc.subcore_barrier()

    one_agg(xr_hbm, p2_hbm, c2_hbm, o2_hbm)
    one_agg(xm_hbm, p3_hbm, c3_hbm, o3_hbm)

  return k


def _make_cnt3():
  SH = 98
  mesh = plsc.VectorSubcoreMesh(core_axis_name="c", subcore_axis_name="s",
                                num_cores=NC, num_subcores=NS)

  @functools.partial(
      pl.kernel,
      out_type=(jax.ShapeDtypeStruct((NR_PAD,), _f32),
                jax.ShapeDtypeStruct((NM_PAD,), _f32),
                jax.ShapeDtypeStruct((NM_PAD,), _f32)),
      mesh=mesh,
      scratch_types=[
          pltpu.VMEM((7 * 2048,), _i32),
          pltpu.VMEM((SH, G), _i32),
          pltpu.VMEM((G,), _f32),
          pltpu.VMEM((2048,), _f32),
          pltpu.VMEM((2048,), _f32),
          pltpu.VMEM((8, 128), _i32),
          pltpu.VMEM_SHARED((C_M + 8,), _f32),
          pltpu.SemaphoreType.DMA,
          pltpu.SemaphoreType.DMA,
      ],
  )
  def k(p1_hbm, c1_hbm, p2_hbm, c2_hbm, p3_hbm, c3_hbm,
        o1_hbm, o2_hbm, o3_hbm,
        pk_v, cdst, ones_v, zb_v, wb_v, cnt2, acc_sp, sem0, sem1):
    c = lax.axis_index("c")
    s = lax.axis_index("s")
    def zi(i, _):
      zb_v[pl.ds(i * 16, 16)] = jnp.zeros((16,), _f32)
      return 0
    lax.fori_loop(0, 2048 // 16, zi, 0)
    def oi(i, _):
      ones_v[pl.ds(i * 16, 16)] = jnp.ones((16,), _f32)
      return 0
    lax.fori_loop(0, G // 16, oi, 0)

    def phase(packed_hbm, counts_hbm, out_hbm, nchunk, C):
      pltpu.sync_copy(counts_hbm, cnt2)
      npass = nchunk // 2
      nw = NS
      wrt = C // nw
      while wrt % 16:
        nw //= 2
        wrt = C // nw
      for p in range(npass):
        kk = c * npass + p
        lo = kk * C
        off0 = pl.multiple_of(s * wrt, 16)
        @pl.when(s < nw)
        def _():
          pltpu.sync_copy(zb_v.at[pl.ds(0, wrt)],
                          acc_sp.at[pl.ds(off0, wrt)])
        plsc.subcore_barrier()

        ck = cnt2[kk, pl.ds(0, 16)][0]
        nblk = lax.shift_right_logical(ck + (G - 1), 7)
        share = lax.shift_right_logical(nblk + (NS - 1), 4)
        wbase = kk * REG + s * share * G
        nst = lax.shift_right_logical(share + 15, 4)
        def st_body(i, _):
          off = pl.multiple_of(wbase + i * 2048, 8)
          pltpu.async_copy(packed_hbm.at[pl.ds(off, 2048)],
                           pk_v.at[pl.ds(i * 2048, 2048)], sem1)
          return 0
        lax.fori_loop(0, nst, st_body, 0)
        def st_drain(i, _):
          off = pl.multiple_of(wbase + i * 2048, 8)
          pltpu.make_async_copy(packed_hbm.at[pl.ds(off, 2048)],
                                pk_v.at[pl.ds(i * 2048, 2048)], sem1).wait()
          return 0
        lax.fori_loop(0, nst, st_drain, 0)
        def up_body(i, _):
          r = i // 8
          cu = (i % 8) * 16
          v = pk_v[pl.ds(i * 16, 16)]
          cdst[r, pl.ds(cu, 16)] = lax.shift_right_logical(v, 17)
          return 0
        lax.fori_loop(0, share * 8, up_body, 0)
        def cb(b, _):
          pltpu.sync_copy(ones_v, acc_sp.at[cdst.at[b]], add=True)
          return 0
        lax.fori_loop(0, share, cb, 0)
        plsc.subcore_barrier()
        ooff = pl.multiple_of(lo + s * wrt, 16)
        @pl.when(s < nw)
        def _():
          pltpu.sync_copy(acc_sp.at[pl.ds(off0, wrt)],
                          wb_v.at[pl.ds(0, wrt)])
          pltpu.sync_copy(wb_v.at[pl.ds(0, wrt)],
                          out_hbm.at[pl.ds(ooff, wrt)])
        plsc.subcore_barrier()

    phase(p1_hbm, c1_hbm, o1_hbm, 2, C_R)
    phase(p2_hbm, c2_hbm, o2_hbm, 8, C_M)
    phase(p3_hbm, c3_hbm, o3_hbm, 8, C_M)

  return k


_sc_cache = {}


def _sc_kernels():
  if "k" not in _sc_cache:
    _sc_cache["k"] = (
        _make_rank(2, C_R),
        _make_rank(8, C_M),
        _make_permute(),
        _make_seg2(NM_PAD, NR_PAD, 2, C_R, False),
        _make_segmm(),
        _make_cnt3(),
    )
  return _sc_cache["k"]


_BLK = 1024


def _dense_r_body(s_ref, cnt_ref, x_ref, wn_ref, wr_ref, b_ref, o_ref):
  agg = s_ref[...] / jnp.maximum(cnt_ref[...], 1.0)
  o_ref[...] = (jnp.dot(agg, wn_ref[...], preferred_element_type=_f32)
                + jnp.dot(x_ref[...], wr_ref[...], preferred_element_type=_f32)
                + b_ref[0:1, :])


def _dense_r(n_pad, s, cnt, x, wn, wr, b):
  grid = (n_pad // _BLK,)
  return pl.pallas_call(
      _dense_r_body,
      grid=grid,
      in_specs=[
          pl.BlockSpec((_BLK, D), lambda i: (i, 0)),
          pl.BlockSpec((_BLK, 1), lambda i: (i, 0)),
          pl.BlockSpec((_BLK, D), lambda i: (i, 0)),
          pl.BlockSpec((D, D), lambda i: (0, 0)),
          pl.BlockSpec((D, D), lambda i: (0, 0)),
          pl.BlockSpec((8, D), lambda i: (0, 0)),
      ],
      out_specs=pl.BlockSpec((_BLK, D), lambda i: (i, 0)),
      out_shape=jax.ShapeDtypeStruct((n_pad, D), _f32),
  )(s, cnt, x, wn, wr, b)


def _dense_m_body(s1_ref, c1_ref, s2_ref, c2_ref, x_ref,
                  wn1_ref, wr1_ref, b1_ref, wn2_ref, wr2_ref, b2_ref, o_ref):
  a1 = s1_ref[...] / jnp.maximum(c1_ref[...], 1.0)
  a2 = s2_ref[...] / jnp.maximum(c2_ref[...], 1.0)
  x = x_ref[...]
  o_ref[...] = (jnp.dot(a1, wn1_ref[...], preferred_element_type=_f32)
                + jnp.dot(x, wr1_ref[...] + wr2_ref[...],
                          preferred_element_type=_f32)
                + jnp.dot(a2, wn2_ref[...], preferred_element_type=_f32)
                + (b1_ref[0:1, :] + b2_ref[0:1, :]))


def _dense_m(n_pad, s1, c1, s2, c2, x, wn1, wr1, b1, wn2, wr2, b2):
  grid = (n_pad // _BLK,)
  row = lambda i: (i, 0)
  full = lambda i: (0, 0)
  return pl.pallas_call(
      _dense_m_body,
      grid=grid,
      in_specs=[
          pl.BlockSpec((_BLK, D), row),
          pl.BlockSpec((_BLK, 1), row),
          pl.BlockSpec((_BLK, D), row),
          pl.BlockSpec((_BLK, 1), row),
          pl.BlockSpec((_BLK, D), row),
          pl.BlockSpec((D, D), full),
          pl.BlockSpec((D, D), full),
          pl.BlockSpec((8, D), full),
          pl.BlockSpec((D, D), full),
          pl.BlockSpec((D, D), full),
          pl.BlockSpec((8, D), full),
      ],
      out_specs=pl.BlockSpec((_BLK, D), row),
      out_shape=jax.ShapeDtypeStruct((n_pad, D), _f32),
  )(s1, c1, s2, c2, x, wn1, wr1, b1, wn2, wr2, b2)


def _head_body(x_ref, emb_ref, ty_ref, wlr_ref, blr_ref,
               w0_ref, w1_ref, bt_ref, o_ref):
  ro = jnp.dot(x_ref[...], wlr_ref[...], preferred_element_type=_f32) \
      + blr_ref[0:1, :]
  emb = emb_ref[...]
  o0 = jnp.dot(emb, w0_ref[...], preferred_element_type=_f32) + bt_ref[0:1, :]
  o1 = jnp.dot(emb, w1_ref[...], preferred_element_type=_f32) + bt_ref[1:2, :]
  sel = jnp.where(ty_ref[...] == 0, o0, o1)
  dot = jnp.sum(ro * sel, axis=1)
  na = jnp.sqrt(jnp.sum(ro * ro, axis=1))
  nb = jnp.sqrt(jnp.sum(sel * sel, axis=1))
  cos = dot / (jnp.maximum(na, 1e-8) * jnp.maximum(nb, 1e-8))
  o_ref[...] = (cos + 1.0) * 0.5


def _head(x, emb, ty, wlr, blr, w0, w1, bt):
  grid = (NR_PAD // _BLK,)
  row = lambda i: (i, 0)
  full = lambda i: (0, 0)
  return pl.pallas_call(
      _head_body,
      grid=grid,
      in_specs=[
          pl.BlockSpec((_BLK, D), row),
          pl.BlockSpec((_BLK, EMB), row),
          pl.BlockSpec((_BLK, 1), row),
          pl.BlockSpec((D, D), full),
          pl.BlockSpec((8, D), full),
          pl.BlockSpec((EMB, D), full),
          pl.BlockSpec((EMB, D), full),
          pl.BlockSpec((8, D), full),
      ],
      out_specs=pl.BlockSpec((_BLK,), lambda i: (i,)),
      out_shape=jax.ShapeDtypeStruct((NR_PAD,), _f32),
  )(x, emb, ty, wlr, blr, w0, w1, bt)


def _pad_bias(b):
  return jnp.pad(b.reshape(1, D), ((0, 7), (0, 0)))


def kernel(x_reaction, x_molecule, ei_m2r, ei_r2m, ei_m2m, output_notes_opt,
           output_nodes_types, Wn, Wr, bconv, W_lr, b_lr, W_types, b_types):
  x_r = jnp.pad(x_reaction, ((0, NR_PAD - N_R), (0, 0)))
  x_m = jnp.pad(x_molecule, ((0, NM_PAD - N_M), (0, 0)))

  def split(ei):
    src = jnp.pad(ei[0].astype(_i32), (0, E_PAD - E))
    dst = jnp.pad(ei[1].astype(_i32), (0, E_PAD - E), constant_values=-1)
    return src, dst

  (_rank_r, _rank_m, _permute, _seg_m2r, _seg_mm, _cnt3) = _sc_kernels()

  s_m2r, d_m2r = split(ei_m2r)
  s_r2m, d_r2m = split(ei_r2m)
  s_m2m, d_m2m = split(ei_m2m)

  rk1, ct1 = _rank_r(d_m2r.reshape(EBLK, 8, 128))
  rk2, ct2 = _rank_m(d_r2m.reshape(EBLK, 8, 128))
  rk3, ct3 = _rank_m(d_m2m.reshape(EBLK, 8, 128))

  p1, p2, p3 = _permute(
      s_m2r, d_m2r, rk1.reshape(NS, ET // G, G),
      s_r2m, d_r2m, rk2.reshape(NS, ET // G, G),
      s_m2m, d_m2m, rk3.reshape(NS, ET // G, G))

  cnt_r, cnt_m1, cnt_m2 = _cnt3(p1, ct1, p2, ct2, p3, ct3)
  cnt_r = cnt_r.reshape(NR_PAD, 1)
  cnt_m1 = cnt_m1.reshape(NM_PAD, 1)
  cnt_m2 = cnt_m2.reshape(NM_PAD, 1)

  for l in range(L):
    agg_r = _seg_m2r(x_m, p1, ct1)
    agg_m1, agg_m2 = _seg_mm(x_r, x_m, p2, ct2, p3, ct3)
    x_r_new = _dense_r(NR_PAD, agg_r, cnt_r, x_r,
                       Wn[l, 0], Wr[l, 0], _pad_bias(bconv[l, 0]))
    x_m_new = _dense_m(NM_PAD, agg_m1, cnt_m1, agg_m2, cnt_m2, x_m,
                       Wn[l, 1], Wr[l, 1], _pad_bias(bconv[l, 1]),
                       Wn[l, 2], Wr[l, 2], _pad_bias(bconv[l, 2]))
    x_r, x_m = x_r_new, x_m_new

  emb = jnp.pad(output_notes_opt, ((0, NR_PAD - N_R), (0, 0)))
  ty = jnp.pad(output_nodes_types.astype(_i32),
               (0, NR_PAD - N_R)).reshape(NR_PAD, 1)
  bt = jnp.pad(b_types, ((0, 6), (0, 0)))
  out = _head(x_r, emb, ty, W_lr, _pad_bias(b_lr),
              W_types[0], W_types[1], bt)
  return out[:N_R]


# speedup vs baseline: 1.2203x; 1.0235x over previous
import functools

import jax
import jax.numpy as jnp
from jax import lax
from jax.experimental import pallas as pl
from jax.experimental.pallas import tpu as pltpu
from jax.experimental.pallas import tpu_sc as plsc

N_R = 10000
N_M = 50000
D = 128
EMB = 1024
L = 4
E = 200000

NR_PAD = 10240
NM_PAD = 50176
E_PAD = 200704
EBLK = E_PAD // 1024

NC = 2
NS = 16
ET = E_PAD // NS
G = 128
REG = 204800
PK = 131072

C_R = NR_PAD // 2
C_M = NM_PAD // 8

_f32 = jnp.float32
_i32 = jnp.int32


def _prefix_8x128(m):
  li = lax.broadcasted_iota(_i32, (8, 128), 1)
  x = m
  for k in (1, 2, 4, 8, 16, 32, 64):
    x = x + jnp.where(li >= k, pltpu.roll(x, k, 1), 0)
  rt = x[:, 127:128]
  si = lax.broadcasted_iota(_i32, (8, 1), 0)
  r = rt
  for k in (1, 2, 4):
    r = r + jnp.where(si >= k, pltpu.roll(r, k, 0), 0)
  return x + (r - rt)


def _make_rank(nchunk, C):
  nreg = nchunk + 1

  def body(d_ref, rank_ref, cnt_ref, carry):
    pid = pl.program_id(0)

    @pl.when(pid == 0)
    def _():
      for k in range(nreg):
        carry[k] = 0

    d = d_ref[0]
    rank = jnp.zeros((8, 128), _i32)
    for k in range(nreg):
      if k < nchunk:
        mk = (d >= k * C) & (d < (k + 1) * C)
      else:
        mk = d < 0
      mi = jnp.where(mk, 1, 0).astype(_i32)
      incl = _prefix_8x128(mi)
      ck = carry[k]
      rank = rank + jnp.where(mk, k * REG + ck + (incl - mi), 0)
      carry[k] = ck + jnp.sum(mi)
    rank_ref[0] = rank

    si = lax.broadcasted_iota(_i32, (8, 128), 0)
    li = lax.broadcasted_iota(_i32, (8, 128), 1)
    cvals = jnp.zeros((8, 128), _i32)
    for k in range(nreg):
      cvals = cvals + jnp.where((si == k) & (li == 0), carry[k], 0)
    cnt_ref[...] = cvals

  def call(d3):
    return pl.pallas_call(
        body,
        grid=(EBLK,),
        in_specs=[pl.BlockSpec((1, 8, 128), lambda i: (i, 0, 0))],
        out_specs=[pl.BlockSpec((1, 8, 128), lambda i: (i, 0, 0)),
                   pl.BlockSpec((8, 128), lambda i: (0, 0))],
        out_shape=[jax.ShapeDtypeStruct((EBLK, 8, 128), _i32),
                   jax.ShapeDtypeStruct((8, 128), _i32)],
        scratch_shapes=[pltpu.SMEM((8,), _i32)],
    )(d3)

  return call


def _make_permute():
  mesh = plsc.VectorSubcoreMesh(core_axis_name="c", subcore_axis_name="s",
                                num_cores=NC, num_subcores=NS)
  nrow = ET // G

  PB = 2048

  @functools.partial(
      pl.kernel,
      out_type=(jax.ShapeDtypeStruct((2 * REG,), _i32),
                jax.ShapeDtypeStruct((2 * REG,), _i32),
                jax.ShapeDtypeStruct((8 * REG,), _i32)),
      mesh=mesh,
      scratch_types=[
          pltpu.VMEM((ET,), _i32),
          pltpu.VMEM((ET,), _i32),
          pltpu.VMEM((nrow, G), _i32),
          pltpu.VMEM((nrow, G), _i32),
          pltpu.VMEM((nrow, G), _i32),
          pltpu.VMEM((PB,), _i32),
          pltpu.VMEM_SHARED((4 * REG + PB,), _i32),
          pltpu.SemaphoreType.DMA,
      ],
  )
  def k(s1, d1, r1, s2, d2, r2, s3, d3, r3, o1, o2, o3,
        src_v, dst_v, rank2, val2, sidx, padb, spbuf, sem):
    c = lax.axis_index("c")
    s = lax.axis_index("s")
    iota16 = lax.iota(_i32, 16)

    def phase(s_hbm, d_hbm, r3_hbm, out_hbm, nchunk, C):
      npc = nchunk // NC
      own = npc * REG
      base_c = c * own
      padval = jnp.full((16,), C * PK, _i32)

      def pfill(i, _):
        padb[pl.ds(i * 16, 16)] = padval
        return 0
      lax.fori_loop(0, PB // 16, pfill, 0)
      shf = own // NS
      def sfill(f, _):
        pltpu.async_copy(padb, spbuf.at[pl.ds(s * shf + f * PB, PB)], sem)
        return 0
      lax.fori_loop(0, shf // PB, sfill, 0)
      def sfill_d(f, _):
        pltpu.make_async_copy(padb, spbuf.at[pl.ds(s * shf + f * PB, PB)],
                              sem).wait()
        return 0
      lax.fori_loop(0, shf // PB, sfill_d, 0)
      if shf % PB:
        pltpu.sync_copy(padb.at[pl.ds(0, shf % PB)],
                        spbuf.at[pl.ds(s * shf + (shf // PB) * PB,
                                       shf % PB)])

      pltpu.sync_copy(s_hbm.at[pl.ds(s * ET, ET)], src_v)
      pltpu.sync_copy(d_hbm.at[pl.ds(s * ET, ET)], dst_v)
      pltpu.sync_copy(r3_hbm.at[s], rank2)

      def pack(i, _):
        r = i // 8
        cc = (i % 8) * 16
        sv = src_v[pl.ds(i * 16, 16)]
        dv = dst_v[pl.ds(i * 16, 16)]
        rk = rank2[r, pl.ds(cc, 16)]
        kk = jnp.zeros((16,), _i32)
        for q in range(1, nchunk):
          kk = kk + jnp.where(dv >= q * C, 1, 0)
        doff = jnp.where(dv >= 0, dv - kk * C, C)
        val2[r, pl.ds(cc, 16)] = sv + doff * PK
        mine = (rk >= base_c) & (rk < base_c + own)
        tr = 4 * REG + lax.bitwise_and(i * 16 + iota16, PB - 1)
        sidx[r, pl.ds(cc, 16)] = jnp.where(mine, rk - base_c, tr)
        return 0
      lax.fori_loop(0, ET // 16, pack, 0)
      plsc.subcore_barrier()

      def scat(j, _):
        pltpu.async_copy(val2.at[j], spbuf.at[sidx.at[j]], sem)
        return 0
      lax.fori_loop(0, nrow, scat, 0)
      def drain(j, _):
        pltpu.make_async_copy(val2.at[j], spbuf.at[sidx.at[j]], sem).wait()
        return 0
      lax.fori_loop(0, nrow, drain, 0)
      plsc.subcore_barrier()

      pltpu.sync_copy(spbuf.at[pl.ds(s * shf, shf)],
                      out_hbm.at[pl.ds(base_c + s * shf, shf)])
      plsc.subcore_barrier()

    phase(s1, d1, r1, o1, 2, C_R)
    phase(s2, d2, r2, o2, 2, C_R)
    phase(s3, d3, r3, o3, 8, C_M)

  return k


def _make_seg2(n_src_pad, n_dst_pad, nchunk, C, counts_mode):
  npass = nchunk // 2
  rpt = C // NS
  n64 = rpt // 64
  rem = rpt - n64 * 64
  mesh = plsc.VectorSubcoreMesh(core_axis_name="c", subcore_axis_name="s",
                                num_cores=NC, num_subcores=NS)

  SH = 98

  @functools.partial(
      pl.kernel,
      out_type=jax.ShapeDtypeStruct((n_dst_pad, D), _f32),
      mesh=mesh,
      scratch_types=[
          pltpu.VMEM((7 * 2048,), _i32),
          pltpu.VMEM((SH, G), _i32),
          pltpu.VMEM((3, G, D), _f32),
          pltpu.VMEM((16, D), _f32),
          pltpu.VMEM((8, 128), _i32),
          pltpu.VMEM_SHARED((C + 8, D), _f32),
          pltpu.SemaphoreType.DMA,
          pltpu.SemaphoreType.DMA,
          pltpu.SemaphoreType.DMA,
          pltpu.SemaphoreType.DMA,
          pltpu.SemaphoreType.DMA,
          pltpu.SemaphoreType.DMA,
          pltpu.SemaphoreType.DMA,
          pltpu.SemaphoreType.DMA,
      ],
  )
  def k(table_hbm, packed_hbm, counts_hbm, out_hbm,
        csrc, cdst, rows_v, zb_v, cnt2, acc_sp, sem0, sem1,
        g0, g1, g2, t0, t1, t2):
    c = lax.axis_index("c")
    s = lax.axis_index("s")
    pltpu.sync_copy(counts_hbm, cnt2)

    def zb_body(i, _):
      r = i // 8
      col = (i % 8) * 16
      zb_v[r, pl.ds(col, 16)] = jnp.zeros((16,), _f32)
      return 0
    lax.fori_loop(0, 16 * 8, zb_body, 0)
    if counts_mode:
      def ob_body(i, _):
        r = i // 8
        col = (i % 8) * 16
        rows_v[0, r, pl.ds(col, 16)] = jnp.ones((16,), _f32)
        return 0
      lax.fori_loop(0, G * 8, ob_body, 0)

    for p in range(npass):
      kk = c * npass + p
      lo = kk * C

      nz = rpt // 16
      rz = rpt - nz * 16
      def z_body(i, _):
        off = pl.multiple_of(s * rpt + i * 16, 8)
        pltpu.async_copy(zb_v, acc_sp.at[pl.ds(off, 16)], sem0)
        return 0
      lax.fori_loop(0, nz, z_body, 0)
      if rz:
        off = pl.multiple_of(s * rpt + nz * 16, 8)
        pltpu.async_copy(zb_v.at[pl.ds(0, rz)], acc_sp.at[pl.ds(off, rz)],
                         sem0)
      def z_drain(i, _):
        off = pl.multiple_of(s * rpt + i * 16, 8)
        pltpu.make_async_copy(zb_v, acc_sp.at[pl.ds(off, 16)], sem0).wait()
        return 0
      lax.fori_loop(0, nz, z_drain, 0)
      if rz:
        off = pl.multiple_of(s * rpt + nz * 16, 8)
        pltpu.make_async_copy(zb_v.at[pl.ds(0, rz)],
                              acc_sp.at[pl.ds(off, rz)], sem0).wait()
      plsc.subcore_barrier()

      ck = cnt2[kk, pl.ds(0, 16)][0]
      nblk = lax.shift_right_logical(ck + (G - 1), 7)
      share = lax.shift_right_logical(nblk + (NS - 1), 4)

      wbase = kk * REG + s * share * G
      nst = lax.shift_right_logical(share + 15, 4)
      def st_body(i, _):
        off = pl.multiple_of(wbase + i * 2048, 8)
        pltpu.async_copy(packed_hbm.at[pl.ds(off, 2048)],
                         csrc.at[pl.ds(i * 2048, 2048)], sem1)
        return 0
      lax.fori_loop(0, nst, st_body, 0)
      def st_drain(i, _):
        off = pl.multiple_of(wbase + i * 2048, 8)
        pltpu.make_async_copy(packed_hbm.at[pl.ds(off, 2048)],
                              csrc.at[pl.ds(i * 2048, 2048)], sem1).wait()
        return 0
      lax.fori_loop(0, nst, st_drain, 0)

      def up_body(i, _):
        r = i // 8
        cu = (i % 8) * 16
        v = csrc[pl.ds(i * 16, 16)]
        if not counts_mode:
          csrc[pl.ds(i * 16, 16)] = lax.bitwise_and(v, PK - 1)
        cdst[r, pl.ds(cu, 16)] = lax.shift_right_logical(v, 17)
        return 0
      lax.fori_loop(0, share * 8, up_body, 0)

      gsem = (g0, g1, g2)
      tsem = (t0, t1, t2)
      def fireg(j, u):
        pltpu.async_copy(table_hbm.at[csrc.at[pl.ds(j * G, G)]],
                         rows_v.at[u], gsem[u])
      def waitg(j, u):
        pltpu.make_async_copy(table_hbm.at[csrc.at[pl.ds(j * G, G)]],
                              rows_v.at[u], gsem[u]).wait()
      def fires(j, u):
        pltpu.async_copy(rows_v.at[u], acc_sp.at[cdst.at[j]], tsem[u],
                         add=True)
      def waits(j, u):
        pltpu.make_async_copy(rows_v.at[u], acc_sp.at[cdst.at[j]],
                              tsem[u]).wait()

      if counts_mode:
        def cb(b, _):
          pltpu.sync_copy(rows_v.at[0], acc_sp.at[cdst.at[b]], add=True)
          return 0
        lax.fori_loop(0, share, cb, 0)
      else:
        def pip(tt, _):
          for u in range(3):
            j = tt * 3 + u
            um = (u + 2) % 3
            @pl.when((j >= 3) & (j - 3 < share))
            def _():
              waits(j - 3, u)
            @pl.when(j < share)
            def _():
              fireg(j, u)
            @pl.when((j >= 1) & (j - 1 < share))
            def _():
              waitg(j - 1, um)
              fires(j - 1, um)
          return 0
        lax.fori_loop(0, lax.div(share + 5, jnp.int32(3)), pip, 0)
      plsc.subcore_barrier()

      def w_body(i, _):
        aoff = pl.multiple_of(s * rpt + i * 64, 8)
        ooff = pl.multiple_of(lo + s * rpt + i * 64, 8)
        pltpu.async_copy(acc_sp.at[pl.ds(aoff, 64)],
                         out_hbm.at[pl.ds(ooff, 64)], sem0)
        return 0
      lax.fori_loop(0, n64, w_body, 0)
      if rem:
        aoff = pl.multiple_of(s * rpt + n64 * 64, 8)
        ooff = pl.multiple_of(lo + s * rpt + n64 * 64, 8)
        pltpu.async_copy(acc_sp.at[pl.ds(aoff, rem)],
                         out_hbm.at[pl.ds(ooff, rem)], sem0)
      def w_drain(i, _):
        aoff = pl.multiple_of(s * rpt + i * 64, 8)
        ooff = pl.multiple_of(lo + s * rpt + i * 64, 8)
        pltpu.make_async_copy(acc_sp.at[pl.ds(aoff, 64)],
                              out_hbm.at[pl.ds(ooff, 64)], sem0).wait()
        return 0
      lax.fori_loop(0, n64, w_drain, 0)
      if rem:
        aoff = pl.multiple_of(s * rpt + n64 * 64, 8)
        ooff = pl.multiple_of(lo + s * rpt + n64 * 64, 8)
        pltpu.make_async_copy(acc_sp.at[pl.ds(aoff, rem)],
                              out_hbm.at[pl.ds(ooff, rem)], sem0).wait()
      plsc.subcore_barrier()

  return k


def _make_segmm():
  mesh = plsc.VectorSubcoreMesh(core_axis_name="c", subcore_axis_name="s",
                                num_cores=NC, num_subcores=NS)

  SH = 98

  @functools.partial(
      pl.kernel,
      out_type=(jax.ShapeDtypeStruct((NR_PAD, D), _f32),
                jax.ShapeDtypeStruct((NM_PAD, D), _f32)),
      mesh=mesh,
      scratch_types=[
          pltpu.VMEM((7 * 2048,), _i32),
          pltpu.VMEM((SH, G), _i32),
          pltpu.VMEM((3, G, D), _f32),
          pltpu.VMEM((16, D), _f32),
          pltpu.VMEM((8, 128), _i32),
          pltpu.VMEM_SHARED((C_M + 8, D), _f32),
          pltpu.SemaphoreType.DMA,
          pltpu.SemaphoreType.DMA,
          pltpu.SemaphoreType.DMA,
          pltpu.SemaphoreType.DMA,
          pltpu.SemaphoreType.DMA,
          pltpu.SemaphoreType.DMA,
          pltpu.SemaphoreType.DMA,
          pltpu.SemaphoreType.DMA,
      ],
  )
  def k(xr_hbm, xm_hbm, p2_hbm, c2_hbm, p3_hbm, c3_hbm, o2_hbm, o3_hbm,
        csrc, cdst, rows_v, zb_v, cnt2, acc_sp, sem0, sem1,
        g0, g1, g2, t0, t1, t2):
    c = lax.axis_index("c")
    s = lax.axis_index("s")

    def zb_body(i, _):
      r = i // 8
      col = (i % 8) * 16
      zb_v[r, pl.ds(col, 16)] = jnp.zeros((16,), _f32)
      return 0
    lax.fori_loop(0, 16 * 8, zb_body, 0)
    def one_agg(table_hbm, packed_hbm, counts_hbm, out_hbm, npass, C):
     pltpu.sync_copy(counts_hbm, cnt2)
     rpt = C // NS
     n64 = rpt // 64
     rem = rpt - n64 * 64
     for p in range(npass):
      kk = c * npass + p
      lo = kk * C

      nz = rpt // 16
      rz = rpt - nz * 16
      def z_body(i, _):
        off = pl.multiple_of(s * rpt + i * 16, 8)
        pltpu.async_copy(zb_v, acc_sp.at[pl.ds(off, 16)], sem0)
        return 0
      lax.fori_loop(0, nz, z_body, 0)
      if rz:
        off = pl.multiple_of(s * rpt + nz * 16, 8)
        pltpu.async_copy(zb_v.at[pl.ds(0, rz)], acc_sp.at[pl.ds(off, rz)],
                         sem0)
      def z_drain(i, _):
        off = pl.multiple_of(s * rpt + i * 16, 8)
        pltpu.make_async_copy(zb_v, acc_sp.at[pl.ds(off, 16)], sem0).wait()
        return 0
      lax.fori_loop(0, nz, z_drain, 0)
      if rz:
        off = pl.multiple_of(s * rpt + nz * 16, 8)
        pltpu.make_async_copy(zb_v.at[pl.ds(0, rz)],
                              acc_sp.at[pl.ds(off, rz)], sem0).wait()
      plsc.subcore_barrier()

      ck = cnt2[kk, pl.ds(0, 16)][0]
      nblk = lax.shift_right_logical(ck + (G - 1), 7)
      share = lax.shift_right_logical(nblk + (NS - 1), 4)

      wbase = kk * REG + s * share * G
      nst = lax.shift_right_logical(share + 15, 4)
      def st_body(i, _):
        off = pl.multiple_of(wbase + i * 2048, 8)
        pltpu.async_copy(packed_hbm.at[pl.ds(off, 2048)],
                         csrc.at[pl.ds(i * 2048, 2048)], sem1)
        return 0
      lax.fori_loop(0, nst, st_body, 0)
      def st_drain(i, _):
        off = pl.multiple_of(wbase + i * 2048, 8)
        pltpu.make_async_copy(packed_hbm.at[pl.ds(off, 2048)],
                              csrc.at[pl.ds(i * 2048, 2048)], sem1).wait()
        return 0
      lax.fori_loop(0, nst, st_drain, 0)

      def up_body(i, _):
        r = i // 8
        cu = (i % 8) * 16
        v = csrc[pl.ds(i * 16, 16)]
        csrc[pl.ds(i * 16, 16)] = lax.bitwise_and(v, PK - 1)
        cdst[r, pl.ds(cu, 16)] = lax.shift_right_logical(v, 17)
        return 0
      lax.fori_loop(0, share * 8, up_body, 0)

      gsem = (g0, g1, g2)
      tsem = (t0, t1, t2)
      def fireg(j, u):
        pltpu.async_copy(table_hbm.at[csrc.at[pl.ds(j * G, G)]],
                         rows_v.at[u], gsem[u])
      def waitg(j, u):
        pltpu.make_async_copy(table_hbm.at[csrc.at[pl.ds(j * G, G)]],
                              rows_v.at[u], gsem[u]).wait()
      def fires(j, u):
        pltpu.async_copy(rows_v.at[u], acc_sp.at[cdst.at[j]], tsem[u],
                         add=True)
      def waits(j, u):
        pltpu.make_async_copy(rows_v.at[u], acc_sp.at[cdst.at[j]],
                              tsem[u]).wait()

      def pip(tt, _):
        for u in range(3):
          j = tt * 3 + u
          um = (u + 2) % 3
          @pl.when((j >= 3) & (j - 3 < share))
          def _():
            waits(j - 3, u)
          @pl.when(j < share)
          def _():
            fireg(j, u)
          @pl.when((j >= 1) & (j - 1 < share))
          def _():
            waitg(j - 1, um)
            fires(j - 1, um)
        return 0
      lax.fori_loop(0, lax.div(share + 5, jnp.int32(3)), pip, 0)
      plsc.subcore_barrier()

      def w_body(i, _):
        aoff = pl.multiple_of(s * rpt + i * 64, 8)
        ooff = pl.multiple_of(lo + s * rpt + i * 64, 8)
        pltpu.async_copy(acc_sp.at[pl.ds(aoff, 64)],
                         out_hbm.at[pl.ds(ooff, 64)], sem0)
        return 0
      lax.fori_loop(0, n64, w_body, 0)
      if rem:
        aoff = pl.multiple_of(s * rpt + n64 * 64, 8)
        ooff = pl.multiple_of(lo + s * rpt + n64 * 64, 8)
        pltpu.async_copy(acc_sp.at[pl.ds(aoff, rem)],
                         out_hbm.at[pl.ds(ooff, rem)], sem0)
      def w_drain(i, _):
        aoff = pl.multiple_of(s * rpt + i * 64, 8)
        ooff = pl.multiple_of(lo + s * rpt + i * 64, 8)
        pltpu.make_async_copy(acc_sp.at[pl.ds(aoff, 64)],
                              out_hbm.at[pl.ds(ooff, 64)], sem0).wait()
        return 0
      lax.fori_loop(0, n64, w_drain, 0)
      if rem:
        aoff = pl.multiple_of(s * rpt + n64 * 64, 8)
        ooff = pl.multiple_of(lo + s * rpt + n64 * 64, 8)
        pltpu.make_async_copy(acc_sp.at[pl.ds(aoff, rem)],
                              out_hbm.at[pl.ds(ooff, rem)], sem0).wait()
      plsc.subcore_barrier()

    one_agg(xr_hbm, p2_hbm, c2_hbm, o2_hbm, 1, C_R)
    one_agg(xm_hbm, p3_hbm, c3_hbm, o3_hbm, 4, C_M)

  return k


def _make_cnt3():
  SH = 98
  mesh = plsc.VectorSubcoreMesh(core_axis_name="c", subcore_axis_name="s",
                                num_cores=NC, num_subcores=NS)

  @functools.partial(
      pl.kernel,
      out_type=(jax.ShapeDtypeStruct((NR_PAD,), _f32),
                jax.ShapeDtypeStruct((NR_PAD,), _f32),
                jax.ShapeDtypeStruct((NM_PAD,), _f32)),
      mesh=mesh,
      scratch_types=[
          pltpu.VMEM((7 * 2048,), _i32),
          pltpu.VMEM((SH, G), _i32),
          pltpu.VMEM((G,), _f32),
          pltpu.VMEM((2048,), _f32),
          pltpu.VMEM((2048,), _f32),
          pltpu.VMEM((8, 128), _i32),
          pltpu.VMEM_SHARED((C_M + 8,), _f32),
          pltpu.SemaphoreType.DMA,
          pltpu.SemaphoreType.DMA,
      ],
  )
  def k(p1_hbm, c1_hbm, p2_hbm, c2_hbm, p3_hbm, c3_hbm,
        o1_hbm, o2_hbm, o3_hbm,
        pk_v, cdst, ones_v, zb_v, wb_v, cnt2, acc_sp, sem0, sem1):
    c = lax.axis_index("c")
    s = lax.axis_index("s")
    def zi(i, _):
      zb_v[pl.ds(i * 16, 16)] = jnp.zeros((16,), _f32)
      return 0
    lax.fori_loop(0, 2048 // 16, zi, 0)
    def oi(i, _):
      ones_v[pl.ds(i * 16, 16)] = jnp.ones((16,), _f32)
      return 0
    lax.fori_loop(0, G // 16, oi, 0)

    def phase(packed_hbm, counts_hbm, out_hbm, nchunk, C):
      pltpu.sync_copy(counts_hbm, cnt2)
      npass = nchunk // 2
      nw = NS
      wrt = C // nw
      while wrt % 16:
        nw //= 2
        wrt = C // nw
      for p in range(npass):
        kk = c * npass + p
        lo = kk * C
        off0 = pl.multiple_of(s * wrt, 16)
        @pl.when(s < nw)
        def _():
          pltpu.sync_copy(zb_v.at[pl.ds(0, wrt)],
                          acc_sp.at[pl.ds(off0, wrt)])
        plsc.subcore_barrier()

        ck = cnt2[kk, pl.ds(0, 16)][0]
        nblk = lax.shift_right_logical(ck + (G - 1), 7)
        share = lax.shift_right_logical(nblk + (NS - 1), 4)
        wbase = kk * REG + s * share * G
        nst = lax.shift_right_logical(share + 15, 4)
        def st_body(i, _):
          off = pl.multiple_of(wbase + i * 2048, 8)
          pltpu.async_copy(packed_hbm.at[pl.ds(off, 2048)],
                           pk_v.at[pl.ds(i * 2048, 2048)], sem1)
          return 0
        lax.fori_loop(0, nst, st_body, 0)
        def st_drain(i, _):
          off = pl.multiple_of(wbase + i * 2048, 8)
          pltpu.make_async_copy(packed_hbm.at[pl.ds(off, 2048)],
                                pk_v.at[pl.ds(i * 2048, 2048)], sem1).wait()
          return 0
        lax.fori_loop(0, nst, st_drain, 0)
        def up_body(i, _):
          r = i // 8
          cu = (i % 8) * 16
          v = pk_v[pl.ds(i * 16, 16)]
          cdst[r, pl.ds(cu, 16)] = lax.shift_right_logical(v, 17)
          return 0
        lax.fori_loop(0, share * 8, up_body, 0)
        def cb(b, _):
          pltpu.sync_copy(ones_v, acc_sp.at[cdst.at[b]], add=True)
          return 0
        lax.fori_loop(0, share, cb, 0)
        plsc.subcore_barrier()
        ooff = pl.multiple_of(lo + s * wrt, 16)
        @pl.when(s < nw)
        def _():
          pltpu.sync_copy(acc_sp.at[pl.ds(off0, wrt)],
                          wb_v.at[pl.ds(0, wrt)])
          pltpu.sync_copy(wb_v.at[pl.ds(0, wrt)],
                          out_hbm.at[pl.ds(ooff, wrt)])
        plsc.subcore_barrier()

    phase(p1_hbm, c1_hbm, o1_hbm, 2, C_R)
    phase(p2_hbm, c2_hbm, o2_hbm, 2, C_R)
    phase(p3_hbm, c3_hbm, o3_hbm, 8, C_M)

  return k


_sc_cache = {}


def _sc_kernels():
  if "k" not in _sc_cache:
    _sc_cache["k"] = (
        _make_rank(2, C_R),
        _make_rank(8, C_M),
        _make_permute(),
        _make_seg2(NM_PAD, NR_PAD, 2, C_R, False),
        _make_segmm(),
        _make_cnt3(),
    )
  return _sc_cache["k"]


_BLK = 1024


def _dense_r_body(s_ref, cnt_ref, x_ref, wn_ref, wr_ref, b_ref, o_ref):
  agg = s_ref[...] / jnp.maximum(cnt_ref[...], 1.0)
  o_ref[...] = (jnp.dot(agg, wn_ref[...], preferred_element_type=_f32)
                + jnp.dot(x_ref[...], wr_ref[...], preferred_element_type=_f32)
                + b_ref[0:1, :])


def _dense_r(n_pad, s, cnt, x, wn, wr, b):
  grid = (n_pad // _BLK,)
  return pl.pallas_call(
      _dense_r_body,
      grid=grid,
      in_specs=[
          pl.BlockSpec((_BLK, D), lambda i: (i, 0)),
          pl.BlockSpec((_BLK, 1), lambda i: (i, 0)),
          pl.BlockSpec((_BLK, D), lambda i: (i, 0)),
          pl.BlockSpec((D, D), lambda i: (0, 0)),
          pl.BlockSpec((D, D), lambda i: (0, 0)),
          pl.BlockSpec((8, D), lambda i: (0, 0)),
      ],
      out_specs=pl.BlockSpec((_BLK, D), lambda i: (i, 0)),
      out_shape=jax.ShapeDtypeStruct((n_pad, D), _f32),
  )(s, cnt, x, wn, wr, b)


def _dense_m_body(s1_ref, c1_ref, s2_ref, c2_ref, x_ref,
                  wn1_ref, wr1_ref, b1_ref, wn2_ref, wr2_ref, b2_ref, o_ref):
  a1 = s1_ref[...] / jnp.maximum(c1_ref[...], 1.0)
  a2 = s2_ref[...] / jnp.maximum(c2_ref[...], 1.0)
  x = x_ref[...]
  m1 = jnp.dot(a1, wn1_ref[...], preferred_element_type=_f32)
  m1 = jnp.where(pl.program_id(0) < NR_PAD // _BLK, m1, 0.0)
  o_ref[...] = (m1
                + jnp.dot(x, wr1_ref[...] + wr2_ref[...],
                          preferred_element_type=_f32)
                + jnp.dot(a2, wn2_ref[...], preferred_element_type=_f32)
                + (b1_ref[0:1, :] + b2_ref[0:1, :]))


def _dense_m(n_pad, s1, c1, s2, c2, x, wn1, wr1, b1, wn2, wr2, b2):
  grid = (n_pad // _BLK,)
  row = lambda i: (i, 0)
  full = lambda i: (0, 0)
  return pl.pallas_call(
      _dense_m_body,
      grid=grid,
      in_specs=[
          pl.BlockSpec((_BLK, D), lambda i: (jnp.minimum(i, NR_PAD // _BLK - 1), 0)),
          pl.BlockSpec((_BLK, 1), lambda i: (jnp.minimum(i, NR_PAD // _BLK - 1), 0)),
          pl.BlockSpec((_BLK, D), row),
          pl.BlockSpec((_BLK, 1), row),
          pl.BlockSpec((_BLK, D), row),
          pl.BlockSpec((D, D), full),
          pl.BlockSpec((D, D), full),
          pl.BlockSpec((8, D), full),
          pl.BlockSpec((D, D), full),
          pl.BlockSpec((D, D), full),
          pl.BlockSpec((8, D), full),
      ],
      out_specs=pl.BlockSpec((_BLK, D), row),
      out_shape=jax.ShapeDtypeStruct((n_pad, D), _f32),
  )(s1, c1, s2, c2, x, wn1, wr1, b1, wn2, wr2, b2)


def _head_body(x_ref, emb_ref, ty_ref, wlr_ref, blr_ref,
               w0_ref, w1_ref, bt_ref, o_ref):
  ro = jnp.dot(x_ref[...], wlr_ref[...], preferred_element_type=_f32) \
      + blr_ref[0:1, :]
  emb = emb_ref[...]
  o0 = jnp.dot(emb, w0_ref[...], preferred_element_type=_f32) + bt_ref[0:1, :]
  o1 = jnp.dot(emb, w1_ref[...], preferred_element_type=_f32) + bt_ref[1:2, :]
  sel = jnp.where(ty_ref[...] == 0, o0, o1)
  dot = jnp.sum(ro * sel, axis=1)
  na = jnp.sqrt(jnp.sum(ro * ro, axis=1))
  nb = jnp.sqrt(jnp.sum(sel * sel, axis=1))
  cos = dot / (jnp.maximum(na, 1e-8) * jnp.maximum(nb, 1e-8))
  o_ref[...] = (cos + 1.0) * 0.5


def _head(x, emb, ty, wlr, blr, w0, w1, bt):
  grid = (NR_PAD // _BLK,)
  row = lambda i: (i, 0)
  full = lambda i: (0, 0)
  return pl.pallas_call(
      _head_body,
      grid=grid,
      in_specs=[
          pl.BlockSpec((_BLK, D), row),
          pl.BlockSpec((_BLK, EMB), row),
          pl.BlockSpec((_BLK, 1), row),
          pl.BlockSpec((D, D), full),
          pl.BlockSpec((8, D), full),
          pl.BlockSpec((EMB, D), full),
          pl.BlockSpec((EMB, D), full),
          pl.BlockSpec((8, D), full),
      ],
      out_specs=pl.BlockSpec((_BLK,), lambda i: (i,)),
      out_shape=jax.ShapeDtypeStruct((NR_PAD,), _f32),
  )(x, emb, ty, wlr, blr, w0, w1, bt)


def _pad_bias(b):
  return jnp.pad(b.reshape(1, D), ((0, 7), (0, 0)))


def kernel(x_reaction, x_molecule, ei_m2r, ei_r2m, ei_m2m, output_notes_opt,
           output_nodes_types, Wn, Wr, bconv, W_lr, b_lr, W_types, b_types):
  x_r = jnp.pad(x_reaction, ((0, NR_PAD - N_R), (0, 0)))
  x_m = jnp.pad(x_molecule, ((0, NM_PAD - N_M), (0, 0)))

  def split(ei):
    src = jnp.pad(ei[0].astype(_i32), (0, E_PAD - E))
    dst = jnp.pad(ei[1].astype(_i32), (0, E_PAD - E), constant_values=-1)
    return src, dst

  (_rank_r, _rank_m, _permute, _seg_m2r, _seg_mm, _cnt3) = _sc_kernels()

  s_m2r, d_m2r = split(ei_m2r)
  s_r2m, d_r2m = split(ei_r2m)
  s_m2m, d_m2m = split(ei_m2m)

  rk1, ct1 = _rank_r(d_m2r.reshape(EBLK, 8, 128))
  rk2, ct2 = _rank_r(d_r2m.reshape(EBLK, 8, 128))
  rk3, ct3 = _rank_m(d_m2m.reshape(EBLK, 8, 128))

  p1, p2, p3 = _permute(
      s_m2r, d_m2r, rk1.reshape(NS, ET // G, G),
      s_r2m, d_r2m, rk2.reshape(NS, ET // G, G),
      s_m2m, d_m2m, rk3.reshape(NS, ET // G, G))

  cnt_r, cnt_m1, cnt_m2 = _cnt3(p1, ct1, p2, ct2, p3, ct3)
  cnt_r = cnt_r.reshape(NR_PAD, 1)
  cnt_m1 = cnt_m1.reshape(NR_PAD, 1)
  cnt_m2 = cnt_m2.reshape(NM_PAD, 1)

  for l in range(L):
    agg_r = _seg_m2r(x_m, p1, ct1)
    agg_m1, agg_m2 = _seg_mm(x_r, x_m, p2, ct2, p3, ct3)
    x_r_new = _dense_r(NR_PAD, agg_r, cnt_r, x_r,
                       Wn[l, 0], Wr[l, 0], _pad_bias(bconv[l, 0]))
    x_m_new = _dense_m(NM_PAD, agg_m1, cnt_m1, agg_m2, cnt_m2, x_m,
                       Wn[l, 1], Wr[l, 1], _pad_bias(bconv[l, 1]),
                       Wn[l, 2], Wr[l, 2], _pad_bias(bconv[l, 2]))
    x_r, x_m = x_r_new, x_m_new

  emb = jnp.pad(output_notes_opt, ((0, NR_PAD - N_R), (0, 0)))
  ty = jnp.pad(output_nodes_types.astype(_i32),
               (0, NR_PAD - N_R)).reshape(NR_PAD, 1)
  bt = jnp.pad(b_types, ((0, 6), (0, 0)))
  out = _head(x_r, emb, ty, W_lr, _pad_bias(b_lr),
              W_types[0], W_types[1], bt)
  return out[:N_R]
